# baseline (speedup 1.0000x reference)
import jax
import jax.numpy as jnp
from jax.experimental import pallas as pl
from jax.experimental.pallas import tpu as pltpu

EMB = 32
SEQ = 16
CONV_OUT = 64
LATENT = 2 * CONV_OUT
ATT_DIM = 100
ATT_PAD = 128
K_CO = 80
K_PAD = 128
NUM_CLASSES = 2
EPS = 1e-7

TM = 2048
_CHUNK_ROWS = TM * SEQ // 128

_GROUPS = ((0, (0, 1, 2, 3, 4, 5)),
           (4, (6, 7, 8, 9)),
           (8, (10, 11, 12, 13, 14)))


def _round_up(a, b):
    return (a + b - 1) // b * b


def _conv_kernel(idx_ref, t0_ref, t1_ref, t2_ref, t3_ref,
                 wg0_ref, wg1_ref, wg2_ref, b_ref, out_ref,
                 idx_smem, sem):
    tbls = (t0_ref, t1_ref, t2_ref, t3_ref)

    cp = pltpu.make_async_copy(idx_ref, idx_smem, sem)
    cp.start()
    cp.wait()

    def gather8(c):
        cols = []
        for g in range(4):
            rows = []
            for m in range(8):
                k0 = m * SEQ + g * 4
                p = (tbls[0][idx_smem[c, k0 + 0]]
                     + tbls[1][idx_smem[c, k0 + 1]])
                q = (tbls[2][idx_smem[c, k0 + 2]]
                     + tbls[3][idx_smem[c, k0 + 3]])
                rows.append(p + q)
            cols.append(jnp.concatenate(rows, axis=0))
        return tuple(cols)

    def conv8(cols):
        x01 = jnp.concatenate(cols[0:2], axis=1)
        x12 = jnp.concatenate(cols[1:3], axis=1)
        x23 = jnp.concatenate(cols[2:4], axis=1)
        ys = (
            jnp.dot(x01, wg0_ref[...], preferred_element_type=jnp.float32),
            jnp.dot(x12, wg1_ref[...], preferred_element_type=jnp.float32),
            jnp.dot(x23, wg2_ref[...], preferred_element_type=jnp.float32),
        )
        chunks = []
        for y, (_, ps) in zip(ys, _GROUPS):
            for i in range(len(ps)):
                chunks.append(y[:, i * LATENT:(i + 1) * LATENT])
        lane = jax.lax.broadcasted_iota(jnp.int32, chunks[-1].shape, 1)
        chunks[-1] = jnp.where(lane < CONV_OUT, chunks[-1], -jnp.inf)
        while len(chunks) > 1:
            nxt = [jnp.maximum(chunks[i], chunks[i + 1])
                   for i in range(0, len(chunks) - 1, 2)]
            if len(chunks) % 2:
                nxt.append(chunks[-1])
            chunks = nxt
        return jnp.maximum(chunks[0] + b_ref[...], 0.0)

    def body(c, carry):
        cols = gather8(c)
        out_ref[pl.ds(pl.multiple_of((c - 1) * 8, 8), 8), :] = conv8(carry)
        return cols

    last = jax.lax.fori_loop(1, _CHUNK_ROWS, body, gather8(0))
    out_ref[pl.ds((_CHUNK_ROWS - 1) * 8, 8), :] = conv8(last)


def _build_group_weights(w1, w2):
    taps = [jnp.concatenate(
        [w1[t] if t < 2 else jnp.zeros((EMB, CONV_OUT), jnp.float32), w2[t]],
        axis=1) for t in range(3)]
    outs = []
    for base, ps in _GROUPS:
        W = jnp.zeros((8 * EMB, len(ps) * LATENT), jnp.float32)
        for i, p in enumerate(ps):
            for t in range(3):
                if p + t >= SEQ:
                    continue
                o = p - base + t
                W = W.at[o * EMB:(o + 1) * EMB, i * LATENT:(i + 1) * LATENT].set(taps[t])
        outs.append(W)
    return outs


def _conv_features(idx, emo_emb, wg, b_fused):
    N = idx.shape[0]
    N_pad = _round_up(N, TM)
    if N_pad != N:
        idx = jnp.pad(idx, ((0, N_pad - N), (0, 0)))
    idx2d = idx.reshape(N_pad * SEQ // 128, 128)

    V = emo_emb.shape[0]
    tbls = []
    for r in range(4):
        t = jnp.zeros((V, 128), jnp.float32).at[:, r * EMB:(r + 1) * EMB].set(emo_emb)
        tbls.append(t.reshape(V, 1, 128))

    out = pl.pallas_call(
        _conv_kernel,
        out_shape=jax.ShapeDtypeStruct((N_pad, LATENT), jnp.float32),
        grid=(N_pad // TM,),
        in_specs=[
            pl.BlockSpec((_CHUNK_ROWS, 128), lambda n: (n, 0)),
            pl.BlockSpec((V, 1, 128), lambda n: (0, 0, 0)),
            pl.BlockSpec((V, 1, 128), lambda n: (0, 0, 0)),
            pl.BlockSpec((V, 1, 128), lambda n: (0, 0, 0)),
            pl.BlockSpec((V, 1, 128), lambda n: (0, 0, 0)),
            pl.BlockSpec(wg[0].shape, lambda n: (0, 0)),
            pl.BlockSpec(wg[1].shape, lambda n: (0, 0)),
            pl.BlockSpec(wg[2].shape, lambda n: (0, 0)),
            pl.BlockSpec((1, LATENT), lambda n: (0, 0)),
        ],
        out_specs=pl.BlockSpec((TM, LATENT), lambda n: (n, 0)),
        scratch_shapes=[
            pltpu.SMEM((_CHUNK_ROWS, 128), jnp.int32),
            pltpu.SemaphoreType.DMA,
        ],
        compiler_params=pltpu.CompilerParams(dimension_semantics=("parallel",)),
    )(idx2d, tbls[0], tbls[1], tbls[2], tbls[3], wg[0], wg[1], wg[2], b_fused)
    return out[:N]


def _softmax_ax1(x):
    m = jnp.max(x, axis=1, keepdims=True)
    e = jnp.exp(x - m)
    return e / jnp.sum(e, axis=1, keepdims=True)


def _head_kernel(xb_ref, xd_ref, Watt_ref, batt_ref, u_ref, Wl_ref, Wc_ref,
                 Ws_ref, whs_ref, whc_ref, Wfc_ref, bfc_ref,
                 preds_ref, As_ref, Ac_ref, ait_ref):
    TB, Ns, _ = xb_ref.shape
    Nc = xd_ref.shape[1]
    XB3 = xb_ref[...]
    XD3 = xd_ref[...]
    XB = XB3.reshape(TB * Ns, LATENT)
    XD = XD3.reshape(TB * Nc, LATENT)
    cn = (((1,), (1,)), ((), ()))

    uit = jnp.tanh(jnp.dot(XD, Watt_ref[...], preferred_element_type=jnp.float32)
                   + batt_ref[...])
    s = jnp.sum(uit.reshape(TB, Nc, LATENT) * u_ref[...].reshape(1, 1, LATENT),
                axis=2, keepdims=True)
    a = jnp.exp(s)
    an = a / (jnp.sum(a, axis=1, keepdims=True) + EPS)
    ait_ref[...] = an
    xd_a = jnp.sum(XD3 * an, axis=1)

    XDWl = jnp.dot(XD, Wl_ref[...], preferred_element_type=jnp.float32)
    Lbig = jax.lax.dot_general(XDWl, XB, cn,
                               preferred_element_type=jnp.float32)
    rb = jax.lax.broadcasted_iota(jnp.int32, Lbig.shape, 0) // Nc
    cb = jax.lax.broadcasted_iota(jnp.int32, Lbig.shape, 1) // Ns
    Lm = jnp.where(rb == cb, jnp.tanh(Lbig), 0.0)

    XBWlT = jax.lax.dot_general(XB, Wl_ref[...], cn,
                                preferred_element_type=jnp.float32)
    LbigT = jax.lax.dot_general(XBWlT, XD, cn,
                                preferred_element_type=jnp.float32)
    rbT = jax.lax.broadcasted_iota(jnp.int32, LbigT.shape, 0) // Ns
    cbT = jax.lax.broadcasted_iota(jnp.int32, LbigT.shape, 1) // Nc
    LmT = jnp.where(rbT == cbT, jnp.tanh(LbigT), 0.0)

    S_b = jax.lax.dot_general(XB, Ws_ref[...], cn,
                              preferred_element_type=jnp.float32)
    C_b = jax.lax.dot_general(XD, Wc_ref[...], cn,
                              preferred_element_type=jnp.float32)
    HsT = jnp.tanh(S_b + jnp.dot(LmT, C_b, preferred_element_type=jnp.float32))
    HcT = jnp.tanh(C_b + jnp.dot(Lm, S_b, preferred_element_type=jnp.float32))

    vs = jnp.sum(HsT * whs_ref[...], axis=1, keepdims=True).reshape(TB, Ns, 1)
    As = _softmax_ax1(vs)
    As_ref[...] = As
    co_s = jnp.sum(XB3 * As, axis=1)

    vc = jnp.sum(HcT * whc_ref[...], axis=1, keepdims=True).reshape(TB, Nc, 1)
    Ac = _softmax_ax1(vc)
    Ac_ref[...] = Ac
    co_c = jnp.sum(XD3 * Ac, axis=1)

    cat = jnp.concatenate([xd_a, co_s, co_c], axis=1)
    logits = (jnp.dot(cat, Wfc_ref[...], preferred_element_type=jnp.float32)
              + bfc_ref[...])
    m = jnp.max(logits, axis=1, keepdims=True)
    e = jnp.exp(logits - m)
    preds_ref[...] = e / jnp.sum(e, axis=1, keepdims=True)


def _head_pass(xb, xd, W_att, b_att, u_att, Wl, Wc, Ws, whs, whc, Wfc, bfc):
    B, Ns, _ = xb.shape
    Nc = xd.shape[1]
    TB = 8
    while B % TB:
        TB //= 2

    W_att_p = jnp.pad(W_att, ((0, 0), (0, ATT_PAD - ATT_DIM)))
    b_att_p = jnp.pad(b_att, ((0, 0), (0, ATT_PAD - ATT_DIM)))
    u_row = jnp.pad(u_att.T, ((0, 0), (0, ATT_PAD - ATT_DIM)))
    Wc_p = jnp.pad(Wc, ((0, K_PAD - K_CO), (0, 0)))
    Ws_p = jnp.pad(Ws, ((0, K_PAD - K_CO), (0, 0)))
    whs_r = jnp.pad(whs, ((0, 0), (0, K_PAD - K_CO)))
    whc_r = jnp.pad(whc, ((0, 0), (0, K_PAD - K_CO)))

    def full(shape):
        return pl.BlockSpec(shape, lambda b, _n=len(shape): (0,) * _n)

    out_shape = (
        jax.ShapeDtypeStruct((B, NUM_CLASSES), jnp.float32),
        jax.ShapeDtypeStruct((B, Ns, 1), jnp.float32),
        jax.ShapeDtypeStruct((B, Nc, 1), jnp.float32),
        jax.ShapeDtypeStruct((B, Nc, 1), jnp.float32),
    )
    in_specs = [
        pl.BlockSpec((TB, Ns, LATENT), lambda b: (b, 0, 0)),
        pl.BlockSpec((TB, Nc, LATENT), lambda b: (b, 0, 0)),
        full((LATENT, ATT_PAD)),
        full((1, ATT_PAD)),
        full((1, ATT_PAD)),
        full((LATENT, LATENT)),
        full((K_PAD, LATENT)),
        full((K_PAD, LATENT)),
        full((1, K_PAD)),
        full((1, K_PAD)),
        full((3 * LATENT, NUM_CLASSES)),
        full((1, NUM_CLASSES)),
    ]
    out_specs = (
        pl.BlockSpec((TB, NUM_CLASSES), lambda b: (b, 0)),
        pl.BlockSpec((TB, Ns, 1), lambda b: (b, 0, 0)),
        pl.BlockSpec((TB, Nc, 1), lambda b: (b, 0, 0)),
        pl.BlockSpec((TB, Nc, 1), lambda b: (b, 0, 0)),
    )
    return pl.pallas_call(
        _head_kernel,
        out_shape=out_shape,
        grid=(B // TB,),
        in_specs=in_specs,
        out_specs=out_specs,
        compiler_params=pltpu.CompilerParams(dimension_semantics=("parallel",)),
    )(xb, xd, W_att_p, b_att_p, u_row, Wl, Wc_p, Ws_p, whs_r, whc_r, Wfc, bfc)


def kernel(content, comments, emo_emb, w1, b1, w2, b2, W_att, b_att, u_att,
           Wl, Wc, Ws, whs, whc, Wfc, bfc):
    B, NS, S = content.shape
    NC = comments.shape[1]

    wg = _build_group_weights(w1, w2)
    b_fused = jnp.concatenate([b1, b2], axis=1)

    xb = _conv_features(content.reshape(B * NS, S), emo_emb, wg,
                        b_fused).reshape(B, NS, LATENT)
    xd = _conv_features(comments.reshape(B * NC, S), emo_emb, wg,
                        b_fused).reshape(B, NC, LATENT)
    return _head_pass(xb, xd, W_att, b_att, u_att, Wl, Wc, Ws, whs, whc, Wfc, bfc)


# speedup vs baseline: 1.6637x; 1.6637x over previous
import jax
import jax.numpy as jnp
from jax.experimental import pallas as pl
from jax.experimental.pallas import tpu as pltpu

EMB = 32
SEQ = 16
CONV_OUT = 64
LATENT = 2 * CONV_OUT
ATT_DIM = 100
ATT_PAD = 128
K_CO = 80
K_PAD = 128
NUM_CLASSES = 2
EPS = 1e-7

TM = 2048
_CHUNK_ROWS = TM * SEQ // 128

_GROUPS = ((0, (0, 1, 2, 3, 4, 5)),
           (4, (6, 7, 8, 9)),
           (8, (10, 11, 12, 13, 14)))


def _round_up(a, b):
    return (a + b - 1) // b * b


def _conv_kernel(idx_ref, t0_ref, t1_ref, t2_ref, t3_ref,
                 wg0_ref, wg1_ref, wg2_ref, b_ref, out_ref,
                 idx_smem, sem):
    tbls = (t0_ref, t1_ref, t2_ref, t3_ref)

    cp = pltpu.make_async_copy(idx_ref, idx_smem, sem)
    cp.start()
    cp.wait()

    def gather8(c):
        cols = []
        for g in range(4):
            rows = []
            for m in range(8):
                k0 = m * SEQ + g * 4
                p = (tbls[0][idx_smem[c, k0 + 0]]
                     + tbls[1][idx_smem[c, k0 + 1]])
                q = (tbls[2][idx_smem[c, k0 + 2]]
                     + tbls[3][idx_smem[c, k0 + 3]])
                rows.append(p + q)
            cols.append(jnp.concatenate(rows, axis=0))
        return tuple(cols)

    def conv_rows(carry):
        x01 = jnp.concatenate(
            [jnp.concatenate(cols[0:2], axis=1) for cols in carry], axis=0)
        x12 = jnp.concatenate(
            [jnp.concatenate(cols[1:3], axis=1) for cols in carry], axis=0)
        x23 = jnp.concatenate(
            [jnp.concatenate(cols[2:4], axis=1) for cols in carry], axis=0)
        ys = (
            jnp.dot(x01, wg0_ref[...], preferred_element_type=jnp.float32),
            jnp.dot(x12, wg1_ref[...], preferred_element_type=jnp.float32),
            jnp.dot(x23, wg2_ref[...], preferred_element_type=jnp.float32),
        )
        chunks = []
        for y, (_, ps) in zip(ys, _GROUPS):
            for i in range(len(ps)):
                chunks.append(y[:, i * LATENT:(i + 1) * LATENT])
        lane = jax.lax.broadcasted_iota(jnp.int32, chunks[-1].shape, 1)
        chunks[-1] = jnp.where(lane < CONV_OUT, chunks[-1], -jnp.inf)
        while len(chunks) > 1:
            nxt = [jnp.maximum(chunks[i], chunks[i + 1])
                   for i in range(0, len(chunks) - 1, 2)]
            if len(chunks) % 2:
                nxt.append(chunks[-1])
            chunks = nxt
        return jnp.maximum(chunks[0] + b_ref[...], 0.0)

    def gather_sc(sc):
        return tuple(gather8(sc * 4 + j) for j in range(4))

    n_sc = _CHUNK_ROWS // 4
    def body(sc, carry):
        new = gather_sc(sc)
        out_ref[pl.ds(pl.multiple_of((sc - 1) * 32, 8), 32), :] = conv_rows(carry)
        return new

    last = jax.lax.fori_loop(1, n_sc, body, gather_sc(0))
    out_ref[pl.ds((n_sc - 1) * 32, 8 * 4), :] = conv_rows(last)


def _build_group_weights(w1, w2):
    taps = [jnp.concatenate(
        [w1[t] if t < 2 else jnp.zeros((EMB, CONV_OUT), jnp.float32), w2[t]],
        axis=1) for t in range(3)]
    outs = []
    for base, ps in _GROUPS:
        W = jnp.zeros((8 * EMB, len(ps) * LATENT), jnp.float32)
        for i, p in enumerate(ps):
            for t in range(3):
                if p + t >= SEQ:
                    continue
                o = p - base + t
                W = W.at[o * EMB:(o + 1) * EMB, i * LATENT:(i + 1) * LATENT].set(taps[t])
        outs.append(W)
    return outs


def _conv_features(idx, emo_emb, wg, b_fused):
    N = idx.shape[0]
    N_pad = _round_up(N, TM)
    if N_pad != N:
        idx = jnp.pad(idx, ((0, N_pad - N), (0, 0)))
    idx2d = idx.reshape(N_pad * SEQ // 128, 128)

    V = emo_emb.shape[0]
    tbls = []
    for r in range(4):
        t = jnp.zeros((V, 128), jnp.float32).at[:, r * EMB:(r + 1) * EMB].set(emo_emb)
        tbls.append(t.reshape(V, 1, 128))

    out = pl.pallas_call(
        _conv_kernel,
        out_shape=jax.ShapeDtypeStruct((N_pad, LATENT), jnp.float32),
        grid=(N_pad // TM,),
        in_specs=[
            pl.BlockSpec((_CHUNK_ROWS, 128), lambda n: (n, 0)),
            pl.BlockSpec((V, 1, 128), lambda n: (0, 0, 0)),
            pl.BlockSpec((V, 1, 128), lambda n: (0, 0, 0)),
            pl.BlockSpec((V, 1, 128), lambda n: (0, 0, 0)),
            pl.BlockSpec((V, 1, 128), lambda n: (0, 0, 0)),
            pl.BlockSpec(wg[0].shape, lambda n: (0, 0)),
            pl.BlockSpec(wg[1].shape, lambda n: (0, 0)),
            pl.BlockSpec(wg[2].shape, lambda n: (0, 0)),
            pl.BlockSpec((1, LATENT), lambda n: (0, 0)),
        ],
        out_specs=pl.BlockSpec((TM, LATENT), lambda n: (n, 0)),
        scratch_shapes=[
            pltpu.SMEM((_CHUNK_ROWS, 128), jnp.int32),
            pltpu.SemaphoreType.DMA,
        ],
        compiler_params=pltpu.CompilerParams(dimension_semantics=("parallel",)),
    )(idx2d, tbls[0], tbls[1], tbls[2], tbls[3], wg[0], wg[1], wg[2], b_fused)
    return out[:N]


def _softmax_ax1(x):
    m = jnp.max(x, axis=1, keepdims=True)
    e = jnp.exp(x - m)
    return e / jnp.sum(e, axis=1, keepdims=True)


def _head_kernel(xb_ref, xd_ref, Watt_ref, batt_ref, u_ref, Wl_ref, Wc_ref,
                 Ws_ref, whs_ref, whc_ref, Wfc_ref, bfc_ref,
                 preds_ref, As_ref, Ac_ref, ait_ref):
    TB, Ns, _ = xb_ref.shape
    Nc = xd_ref.shape[1]
    XB3 = xb_ref[...]
    XD3 = xd_ref[...]
    XB = XB3.reshape(TB * Ns, LATENT)
    XD = XD3.reshape(TB * Nc, LATENT)
    cn = (((1,), (1,)), ((), ()))

    uit = jnp.tanh(jnp.dot(XD, Watt_ref[...], preferred_element_type=jnp.float32)
                   + batt_ref[...])
    s = jnp.sum(uit.reshape(TB, Nc, LATENT) * u_ref[...].reshape(1, 1, LATENT),
                axis=2, keepdims=True)
    a = jnp.exp(s)
    an = a / (jnp.sum(a, axis=1, keepdims=True) + EPS)
    ait_ref[...] = an
    xd_a = jnp.sum(XD3 * an, axis=1)

    XDWl = jnp.dot(XD, Wl_ref[...], preferred_element_type=jnp.float32)
    Lbig = jax.lax.dot_general(XDWl, XB, cn,
                               preferred_element_type=jnp.float32)
    rb = jax.lax.broadcasted_iota(jnp.int32, Lbig.shape, 0) // Nc
    cb = jax.lax.broadcasted_iota(jnp.int32, Lbig.shape, 1) // Ns
    Lm = jnp.where(rb == cb, jnp.tanh(Lbig), 0.0)

    XBWlT = jax.lax.dot_general(XB, Wl_ref[...], cn,
                                preferred_element_type=jnp.float32)
    LbigT = jax.lax.dot_general(XBWlT, XD, cn,
                                preferred_element_type=jnp.float32)
    rbT = jax.lax.broadcasted_iota(jnp.int32, LbigT.shape, 0) // Ns
    cbT = jax.lax.broadcasted_iota(jnp.int32, LbigT.shape, 1) // Nc
    LmT = jnp.where(rbT == cbT, jnp.tanh(LbigT), 0.0)

    S_b = jax.lax.dot_general(XB, Ws_ref[...], cn,
                              preferred_element_type=jnp.float32)
    C_b = jax.lax.dot_general(XD, Wc_ref[...], cn,
                              preferred_element_type=jnp.float32)
    HsT = jnp.tanh(S_b + jnp.dot(LmT, C_b, preferred_element_type=jnp.float32))
    HcT = jnp.tanh(C_b + jnp.dot(Lm, S_b, preferred_element_type=jnp.float32))

    vs = jnp.sum(HsT * whs_ref[...], axis=1, keepdims=True).reshape(TB, Ns, 1)
    As = _softmax_ax1(vs)
    As_ref[...] = As
    co_s = jnp.sum(XB3 * As, axis=1)

    vc = jnp.sum(HcT * whc_ref[...], axis=1, keepdims=True).reshape(TB, Nc, 1)
    Ac = _softmax_ax1(vc)
    Ac_ref[...] = Ac
    co_c = jnp.sum(XD3 * Ac, axis=1)

    cat = jnp.concatenate([xd_a, co_s, co_c], axis=1)
    logits = (jnp.dot(cat, Wfc_ref[...], preferred_element_type=jnp.float32)
              + bfc_ref[...])
    m = jnp.max(logits, axis=1, keepdims=True)
    e = jnp.exp(logits - m)
    preds_ref[...] = e / jnp.sum(e, axis=1, keepdims=True)


def _head_pass(xb, xd, W_att, b_att, u_att, Wl, Wc, Ws, whs, whc, Wfc, bfc):
    B, Ns, _ = xb.shape
    Nc = xd.shape[1]
    TB = 8
    while B % TB:
        TB //= 2

    W_att_p = jnp.pad(W_att, ((0, 0), (0, ATT_PAD - ATT_DIM)))
    b_att_p = jnp.pad(b_att, ((0, 0), (0, ATT_PAD - ATT_DIM)))
    u_row = jnp.pad(u_att.T, ((0, 0), (0, ATT_PAD - ATT_DIM)))
    Wc_p = jnp.pad(Wc, ((0, K_PAD - K_CO), (0, 0)))
    Ws_p = jnp.pad(Ws, ((0, K_PAD - K_CO), (0, 0)))
    whs_r = jnp.pad(whs, ((0, 0), (0, K_PAD - K_CO)))
    whc_r = jnp.pad(whc, ((0, 0), (0, K_PAD - K_CO)))

    def full(shape):
        return pl.BlockSpec(shape, lambda b, _n=len(shape): (0,) * _n)

    out_shape = (
        jax.ShapeDtypeStruct((B, NUM_CLASSES), jnp.float32),
        jax.ShapeDtypeStruct((B, Ns, 1), jnp.float32),
        jax.ShapeDtypeStruct((B, Nc, 1), jnp.float32),
        jax.ShapeDtypeStruct((B, Nc, 1), jnp.float32),
    )
    in_specs = [
        pl.BlockSpec((TB, Ns, LATENT), lambda b: (b, 0, 0)),
        pl.BlockSpec((TB, Nc, LATENT), lambda b: (b, 0, 0)),
        full((LATENT, ATT_PAD)),
        full((1, ATT_PAD)),
        full((1, ATT_PAD)),
        full((LATENT, LATENT)),
        full((K_PAD, LATENT)),
        full((K_PAD, LATENT)),
        full((1, K_PAD)),
        full((1, K_PAD)),
        full((3 * LATENT, NUM_CLASSES)),
        full((1, NUM_CLASSES)),
    ]
    out_specs = (
        pl.BlockSpec((TB, NUM_CLASSES), lambda b: (b, 0)),
        pl.BlockSpec((TB, Ns, 1), lambda b: (b, 0, 0)),
        pl.BlockSpec((TB, Nc, 1), lambda b: (b, 0, 0)),
        pl.BlockSpec((TB, Nc, 1), lambda b: (b, 0, 0)),
    )
    return pl.pallas_call(
        _head_kernel,
        out_shape=out_shape,
        grid=(B // TB,),
        in_specs=in_specs,
        out_specs=out_specs,
        compiler_params=pltpu.CompilerParams(dimension_semantics=("parallel",)),
    )(xb, xd, W_att_p, b_att_p, u_row, Wl, Wc_p, Ws_p, whs_r, whc_r, Wfc, bfc)


def kernel(content, comments, emo_emb, w1, b1, w2, b2, W_att, b_att, u_att,
           Wl, Wc, Ws, whs, whc, Wfc, bfc):
    B, NS, S = content.shape
    NC = comments.shape[1]

    wg = _build_group_weights(w1, w2)
    b_fused = jnp.concatenate([b1, b2], axis=1)

    xb = _conv_features(content.reshape(B * NS, S), emo_emb, wg,
                        b_fused).reshape(B, NS, LATENT)
    xd = _conv_features(comments.reshape(B * NC, S), emo_emb, wg,
                        b_fused).reshape(B, NC, LATENT)
    return _head_pass(xb, xd, W_att, b_att, u_att, Wl, Wc, Ws, whs, whc, Wfc, bfc)


# speedup vs baseline: 1.9890x; 1.1955x over previous
import jax
import jax.numpy as jnp
from jax.experimental import pallas as pl
from jax.experimental.pallas import tpu as pltpu

EMB = 32
SEQ = 16
CONV_OUT = 64
LATENT = 2 * CONV_OUT
ATT_DIM = 100
ATT_PAD = 128
K_CO = 80
K_PAD = 128
NUM_CLASSES = 2
EPS = 1e-7

TM = 1024
_CHUNK_ROWS = TM * SEQ // 128

_GROUPS = ((0, (0, 1, 2, 3, 4, 5)),
           (4, (6, 7, 8, 9)),
           (8, (10, 11, 12, 13, 14)))


def _round_up(a, b):
    return (a + b - 1) // b * b


def _conv_kernel(idx_ref, t0_ref, t1_ref, t2_ref, t3_ref,
                 wg0_ref, wg1_ref, wg2_ref, b_ref, out_ref,
                 x_scratch, idx_smem, sem):
    tbls = (t0_ref, t1_ref, t2_ref, t3_ref)

    cp = pltpu.make_async_copy(idx_ref, idx_smem, sem)
    cp.start()
    cp.wait()

    def chunk(c, _):
        cols = []
        for g in range(4):
            rows = []
            for m in range(8):
                k0 = m * SEQ + g * 4
                p = (tbls[0][idx_smem[c, k0 + 0]]
                     + tbls[1][idx_smem[c, k0 + 1]])
                q = (tbls[2][idx_smem[c, k0 + 2]]
                     + tbls[3][idx_smem[c, k0 + 3]])
                rows.append(p + q)
            cols.append(jnp.concatenate(rows, axis=0))
        r0 = pl.multiple_of(c * 8, 8)
        for g in range(4):
            x_scratch[pl.ds(r0, 8), g * 128:(g + 1) * 128] = cols[g]
        return _

    jax.lax.fori_loop(0, _CHUNK_ROWS, chunk, 0)

    x = x_scratch[...]
    ys = (
        jnp.dot(x[:, 0:256], wg0_ref[...], preferred_element_type=jnp.float32),
        jnp.dot(x[:, 128:384], wg1_ref[...], preferred_element_type=jnp.float32),
        jnp.dot(x[:, 256:512], wg2_ref[...], preferred_element_type=jnp.float32),
    )
    chunks = []
    for y, (_, ps) in zip(ys, _GROUPS):
        for i in range(len(ps)):
            chunks.append(y[:, i * LATENT:(i + 1) * LATENT])
    lane = jax.lax.broadcasted_iota(jnp.int32, chunks[-1].shape, 1)
    chunks[-1] = jnp.where(lane < CONV_OUT, chunks[-1], -jnp.inf)
    while len(chunks) > 1:
        nxt = [jnp.maximum(chunks[i], chunks[i + 1])
               for i in range(0, len(chunks) - 1, 2)]
        if len(chunks) % 2:
            nxt.append(chunks[-1])
        chunks = nxt
    out_ref[...] = jnp.maximum(chunks[0] + b_ref[...], 0.0)


def _build_group_weights(w1, w2):
    taps = [jnp.concatenate(
        [w1[t] if t < 2 else jnp.zeros((EMB, CONV_OUT), jnp.float32), w2[t]],
        axis=1) for t in range(3)]
    outs = []
    for base, ps in _GROUPS:
        W = jnp.zeros((8 * EMB, len(ps) * LATENT), jnp.float32)
        for i, p in enumerate(ps):
            for t in range(3):
                if p + t >= SEQ:
                    continue
                o = p - base + t
                W = W.at[o * EMB:(o + 1) * EMB, i * LATENT:(i + 1) * LATENT].set(taps[t])
        outs.append(W)
    return outs


def _conv_features(idx, emo_emb, wg, b_fused):
    N = idx.shape[0]
    N_pad = _round_up(N, TM)
    if N_pad != N:
        idx = jnp.pad(idx, ((0, N_pad - N), (0, 0)))
    idx2d = idx.reshape(N_pad * SEQ // 128, 128)

    V = emo_emb.shape[0]
    tbls = []
    for r in range(4):
        t = jnp.zeros((V, 128), jnp.float32).at[:, r * EMB:(r + 1) * EMB].set(emo_emb)
        tbls.append(t.reshape(V, 1, 128))

    out = pl.pallas_call(
        _conv_kernel,
        out_shape=jax.ShapeDtypeStruct((N_pad, LATENT), jnp.float32),
        grid=(N_pad // TM,),
        in_specs=[
            pl.BlockSpec((_CHUNK_ROWS, 128), lambda n: (n, 0)),
            pl.BlockSpec((V, 1, 128), lambda n: (0, 0, 0)),
            pl.BlockSpec((V, 1, 128), lambda n: (0, 0, 0)),
            pl.BlockSpec((V, 1, 128), lambda n: (0, 0, 0)),
            pl.BlockSpec((V, 1, 128), lambda n: (0, 0, 0)),
            pl.BlockSpec(wg[0].shape, lambda n: (0, 0)),
            pl.BlockSpec(wg[1].shape, lambda n: (0, 0)),
            pl.BlockSpec(wg[2].shape, lambda n: (0, 0)),
            pl.BlockSpec((1, LATENT), lambda n: (0, 0)),
        ],
        out_specs=pl.BlockSpec((TM, LATENT), lambda n: (n, 0)),
        scratch_shapes=[
            pltpu.VMEM((TM, SEQ * EMB), jnp.float32),
            pltpu.SMEM((_CHUNK_ROWS, 128), jnp.int32),
            pltpu.SemaphoreType.DMA,
        ],
        compiler_params=pltpu.CompilerParams(dimension_semantics=("parallel",)),
    )(idx2d, tbls[0], tbls[1], tbls[2], tbls[3], wg[0], wg[1], wg[2], b_fused)
    return out[:N]


def _softmax_ax1(x):
    m = jnp.max(x, axis=1, keepdims=True)
    e = jnp.exp(x - m)
    return e / jnp.sum(e, axis=1, keepdims=True)


def _head_kernel(xb_ref, xd_ref, Watt_ref, batt_ref, u_ref, Wl_ref, Wc_ref,
                 Ws_ref, whs_ref, whc_ref, Wfc_ref, bfc_ref,
                 preds_ref, As_ref, Ac_ref, ait_ref):
    TB, Ns, _ = xb_ref.shape
    Nc = xd_ref.shape[1]
    XB3 = xb_ref[...]
    XD3 = xd_ref[...]
    XB = XB3.reshape(TB * Ns, LATENT)
    XD = XD3.reshape(TB * Nc, LATENT)
    cn = (((1,), (1,)), ((), ()))

    uit = jnp.tanh(jnp.dot(XD, Watt_ref[...], preferred_element_type=jnp.float32)
                   + batt_ref[...])
    s = jnp.sum(uit.reshape(TB, Nc, LATENT) * u_ref[...].reshape(1, 1, LATENT),
                axis=2, keepdims=True)
    a = jnp.exp(s)
    an = a / (jnp.sum(a, axis=1, keepdims=True) + EPS)
    ait_ref[...] = an
    xd_a = jnp.sum(XD3 * an, axis=1)

    XDWl = jnp.dot(XD, Wl_ref[...], preferred_element_type=jnp.float32)
    Lbig = jax.lax.dot_general(XDWl, XB, cn,
                               preferred_element_type=jnp.float32)
    rb = jax.lax.broadcasted_iota(jnp.int32, Lbig.shape, 0) // Nc
    cb = jax.lax.broadcasted_iota(jnp.int32, Lbig.shape, 1) // Ns
    Lm = jnp.where(rb == cb, jnp.tanh(Lbig), 0.0)

    XBWlT = jax.lax.dot_general(XB, Wl_ref[...], cn,
                                preferred_element_type=jnp.float32)
    LbigT = jax.lax.dot_general(XBWlT, XD, cn,
                                preferred_element_type=jnp.float32)
    rbT = jax.lax.broadcasted_iota(jnp.int32, LbigT.shape, 0) // Ns
    cbT = jax.lax.broadcasted_iota(jnp.int32, LbigT.shape, 1) // Nc
    LmT = jnp.where(rbT == cbT, jnp.tanh(LbigT), 0.0)

    S_b = jax.lax.dot_general(XB, Ws_ref[...], cn,
                              preferred_element_type=jnp.float32)
    C_b = jax.lax.dot_general(XD, Wc_ref[...], cn,
                              preferred_element_type=jnp.float32)
    HsT = jnp.tanh(S_b + jnp.dot(LmT, C_b, preferred_element_type=jnp.float32))
    HcT = jnp.tanh(C_b + jnp.dot(Lm, S_b, preferred_element_type=jnp.float32))

    vs = jnp.sum(HsT * whs_ref[...], axis=1, keepdims=True).reshape(TB, Ns, 1)
    As = _softmax_ax1(vs)
    As_ref[...] = As
    co_s = jnp.sum(XB3 * As, axis=1)

    vc = jnp.sum(HcT * whc_ref[...], axis=1, keepdims=True).reshape(TB, Nc, 1)
    Ac = _softmax_ax1(vc)
    Ac_ref[...] = Ac
    co_c = jnp.sum(XD3 * Ac, axis=1)

    cat = jnp.concatenate([xd_a, co_s, co_c], axis=1)
    logits = (jnp.dot(cat, Wfc_ref[...], preferred_element_type=jnp.float32)
              + bfc_ref[...])
    m = jnp.max(logits, axis=1, keepdims=True)
    e = jnp.exp(logits - m)
    preds_ref[...] = e / jnp.sum(e, axis=1, keepdims=True)


def _head_pass(xb, xd, W_att, b_att, u_att, Wl, Wc, Ws, whs, whc, Wfc, bfc):
    B, Ns, _ = xb.shape
    Nc = xd.shape[1]
    TB = 16
    while B % TB:
        TB //= 2

    W_att_p = jnp.pad(W_att, ((0, 0), (0, ATT_PAD - ATT_DIM)))
    b_att_p = jnp.pad(b_att, ((0, 0), (0, ATT_PAD - ATT_DIM)))
    u_row = jnp.pad(u_att.T, ((0, 0), (0, ATT_PAD - ATT_DIM)))
    Wc_p = jnp.pad(Wc, ((0, K_PAD - K_CO), (0, 0)))
    Ws_p = jnp.pad(Ws, ((0, K_PAD - K_CO), (0, 0)))
    whs_r = jnp.pad(whs, ((0, 0), (0, K_PAD - K_CO)))
    whc_r = jnp.pad(whc, ((0, 0), (0, K_PAD - K_CO)))

    def full(shape):
        return pl.BlockSpec(shape, lambda b, _n=len(shape): (0,) * _n)

    out_shape = (
        jax.ShapeDtypeStruct((B, NUM_CLASSES), jnp.float32),
        jax.ShapeDtypeStruct((B, Ns, 1), jnp.float32),
        jax.ShapeDtypeStruct((B, Nc, 1), jnp.float32),
        jax.ShapeDtypeStruct((B, Nc, 1), jnp.float32),
    )
    in_specs = [
        pl.BlockSpec((TB, Ns, LATENT), lambda b: (b, 0, 0)),
        pl.BlockSpec((TB, Nc, LATENT), lambda b: (b, 0, 0)),
        full((LATENT, ATT_PAD)),
        full((1, ATT_PAD)),
        full((1, ATT_PAD)),
        full((LATENT, LATENT)),
        full((K_PAD, LATENT)),
        full((K_PAD, LATENT)),
        full((1, K_PAD)),
        full((1, K_PAD)),
        full((3 * LATENT, NUM_CLASSES)),
        full((1, NUM_CLASSES)),
    ]
    out_specs = (
        pl.BlockSpec((TB, NUM_CLASSES), lambda b: (b, 0)),
        pl.BlockSpec((TB, Ns, 1), lambda b: (b, 0, 0)),
        pl.BlockSpec((TB, Nc, 1), lambda b: (b, 0, 0)),
        pl.BlockSpec((TB, Nc, 1), lambda b: (b, 0, 0)),
    )
    return pl.pallas_call(
        _head_kernel,
        out_shape=out_shape,
        grid=(B // TB,),
        in_specs=in_specs,
        out_specs=out_specs,
        compiler_params=pltpu.CompilerParams(dimension_semantics=("parallel",)),
    )(xb, xd, W_att_p, b_att_p, u_row, Wl, Wc_p, Ws_p, whs_r, whc_r, Wfc, bfc)


def kernel(content, comments, emo_emb, w1, b1, w2, b2, W_att, b_att, u_att,
           Wl, Wc, Ws, whs, whc, Wfc, bfc):
    B, NS, S = content.shape
    NC = comments.shape[1]

    wg = _build_group_weights(w1, w2)
    b_fused = jnp.concatenate([b1, b2], axis=1)

    xb = _conv_features(content.reshape(B * NS, S), emo_emb, wg,
                        b_fused).reshape(B, NS, LATENT)
    xd = _conv_features(comments.reshape(B * NC, S), emo_emb, wg,
                        b_fused).reshape(B, NC, LATENT)
    return _head_pass(xb, xd, W_att, b_att, u_att, Wl, Wc, Ws, whs, whc, Wfc, bfc)


# speedup vs baseline: 2.0331x; 1.0221x over previous
import jax
import jax.numpy as jnp
from jax.experimental import pallas as pl
from jax.experimental.pallas import tpu as pltpu

EMB = 32
SEQ = 16
CONV_OUT = 64
LATENT = 2 * CONV_OUT
ATT_DIM = 100
ATT_PAD = 128
K_CO = 80
K_PAD = 128
NUM_CLASSES = 2
EPS = 1e-7

TM = 1024
_CHUNK_ROWS = TM * SEQ // 128

_GROUPS = ((0, (0, 1, 2, 3, 4, 5)),
           (4, (6, 7, 8, 9)),
           (8, (10, 11, 12, 13, 14)))


def _round_up(a, b):
    return (a + b - 1) // b * b


def _conv_kernel(idx_ref, t0_ref, t1_ref, t2_ref, t3_ref,
                 wg0_ref, wg1_ref, wg2_ref, b_ref, out_ref,
                 x_scratch, idx_smem, sem):
    tbls = (t0_ref, t1_ref, t2_ref, t3_ref)

    cp = pltpu.make_async_copy(idx_ref, idx_smem, sem)
    cp.start()
    cp.wait()

    def chunk(c2, _):
        for dc in range(2):
            c = c2 * 2 + dc
            cols = []
            for g in range(4):
                rows = []
                for m in range(8):
                    k0 = m * SEQ + g * 4
                    p = (tbls[0][idx_smem[c, k0 + 0]]
                         + tbls[1][idx_smem[c, k0 + 1]])
                    q = (tbls[2][idx_smem[c, k0 + 2]]
                         + tbls[3][idx_smem[c, k0 + 3]])
                    rows.append(p + q)
                cols.append(jnp.concatenate(rows, axis=0))
            r0 = pl.multiple_of(c * 8, 8)
            for g in range(4):
                x_scratch[pl.ds(r0, 8), g * 128:(g + 1) * 128] = cols[g]
        return _

    jax.lax.fori_loop(0, _CHUNK_ROWS // 2, chunk, 0)

    x = x_scratch[...]
    ys = (
        jnp.dot(x[:, 0:256], wg0_ref[...], preferred_element_type=jnp.float32),
        jnp.dot(x[:, 128:384], wg1_ref[...], preferred_element_type=jnp.float32),
        jnp.dot(x[:, 256:512], wg2_ref[...], preferred_element_type=jnp.float32),
    )
    chunks = []
    for y, (_, ps) in zip(ys, _GROUPS):
        for i in range(len(ps)):
            chunks.append(y[:, i * LATENT:(i + 1) * LATENT])
    lane = jax.lax.broadcasted_iota(jnp.int32, chunks[-1].shape, 1)
    chunks[-1] = jnp.where(lane < CONV_OUT, chunks[-1], -jnp.inf)
    while len(chunks) > 1:
        nxt = [jnp.maximum(chunks[i], chunks[i + 1])
               for i in range(0, len(chunks) - 1, 2)]
        if len(chunks) % 2:
            nxt.append(chunks[-1])
        chunks = nxt
    out_ref[...] = jnp.maximum(chunks[0] + b_ref[...], 0.0)


def _build_group_weights(w1, w2):
    taps = [jnp.concatenate(
        [w1[t] if t < 2 else jnp.zeros((EMB, CONV_OUT), jnp.float32), w2[t]],
        axis=1) for t in range(3)]
    outs = []
    for base, ps in _GROUPS:
        W = jnp.zeros((8 * EMB, len(ps) * LATENT), jnp.float32)
        for i, p in enumerate(ps):
            for t in range(3):
                if p + t >= SEQ:
                    continue
                o = p - base + t
                W = W.at[o * EMB:(o + 1) * EMB, i * LATENT:(i + 1) * LATENT].set(taps[t])
        outs.append(W)
    return outs


def _conv_features(idx, emo_emb, wg, b_fused):
    N = idx.shape[0]
    N_pad = _round_up(N, TM)
    if N_pad != N:
        idx = jnp.pad(idx, ((0, N_pad - N), (0, 0)))
    idx2d = idx.reshape(N_pad * SEQ // 128, 128)

    V = emo_emb.shape[0]
    tbls = []
    for r in range(4):
        t = jnp.zeros((V, 128), jnp.float32).at[:, r * EMB:(r + 1) * EMB].set(emo_emb)
        tbls.append(t.reshape(V, 1, 128))

    out = pl.pallas_call(
        _conv_kernel,
        out_shape=jax.ShapeDtypeStruct((N_pad, LATENT), jnp.float32),
        grid=(N_pad // TM,),
        in_specs=[
            pl.BlockSpec((_CHUNK_ROWS, 128), lambda n: (n, 0)),
            pl.BlockSpec((V, 1, 128), lambda n: (0, 0, 0)),
            pl.BlockSpec((V, 1, 128), lambda n: (0, 0, 0)),
            pl.BlockSpec((V, 1, 128), lambda n: (0, 0, 0)),
            pl.BlockSpec((V, 1, 128), lambda n: (0, 0, 0)),
            pl.BlockSpec(wg[0].shape, lambda n: (0, 0)),
            pl.BlockSpec(wg[1].shape, lambda n: (0, 0)),
            pl.BlockSpec(wg[2].shape, lambda n: (0, 0)),
            pl.BlockSpec((1, LATENT), lambda n: (0, 0)),
        ],
        out_specs=pl.BlockSpec((TM, LATENT), lambda n: (n, 0)),
        scratch_shapes=[
            pltpu.VMEM((TM, SEQ * EMB), jnp.float32),
            pltpu.SMEM((_CHUNK_ROWS, 128), jnp.int32),
            pltpu.SemaphoreType.DMA,
        ],
        compiler_params=pltpu.CompilerParams(dimension_semantics=("parallel",)),
    )(idx2d, tbls[0], tbls[1], tbls[2], tbls[3], wg[0], wg[1], wg[2], b_fused)
    return out[:N]


def _softmax_ax1(x):
    m = jnp.max(x, axis=1, keepdims=True)
    e = jnp.exp(x - m)
    return e / jnp.sum(e, axis=1, keepdims=True)


def _head_kernel(xb_ref, xd_ref, Watt_ref, batt_ref, u_ref, Wl_ref, Wc_ref,
                 Ws_ref, whs_ref, whc_ref, Wfc_ref, bfc_ref,
                 preds_ref, As_ref, Ac_ref, ait_ref):
    TB, Ns, _ = xb_ref.shape
    Nc = xd_ref.shape[1]
    XB3 = xb_ref[...]
    XD3 = xd_ref[...]
    XB = XB3.reshape(TB * Ns, LATENT)
    XD = XD3.reshape(TB * Nc, LATENT)
    cn = (((1,), (1,)), ((), ()))

    uit = jnp.tanh(jnp.dot(XD, Watt_ref[...], preferred_element_type=jnp.float32)
                   + batt_ref[...])
    s = jnp.sum(uit.reshape(TB, Nc, LATENT) * u_ref[...].reshape(1, 1, LATENT),
                axis=2, keepdims=True)
    a = jnp.exp(s)
    an = a / (jnp.sum(a, axis=1, keepdims=True) + EPS)
    ait_ref[...] = an
    xd_a = jnp.sum(XD3 * an, axis=1)

    XDWl = jnp.dot(XD, Wl_ref[...], preferred_element_type=jnp.float32)
    Lbig = jax.lax.dot_general(XDWl, XB, cn,
                               preferred_element_type=jnp.float32)
    rb = jax.lax.broadcasted_iota(jnp.int32, Lbig.shape, 0) // Nc
    cb = jax.lax.broadcasted_iota(jnp.int32, Lbig.shape, 1) // Ns
    Lm = jnp.where(rb == cb, jnp.tanh(Lbig), 0.0)

    XBWlT = jax.lax.dot_general(XB, Wl_ref[...], cn,
                                preferred_element_type=jnp.float32)
    LbigT = jax.lax.dot_general(XBWlT, XD, cn,
                                preferred_element_type=jnp.float32)
    rbT = jax.lax.broadcasted_iota(jnp.int32, LbigT.shape, 0) // Ns
    cbT = jax.lax.broadcasted_iota(jnp.int32, LbigT.shape, 1) // Nc
    LmT = jnp.where(rbT == cbT, jnp.tanh(LbigT), 0.0)

    S_b = jax.lax.dot_general(XB, Ws_ref[...], cn,
                              preferred_element_type=jnp.float32)
    C_b = jax.lax.dot_general(XD, Wc_ref[...], cn,
                              preferred_element_type=jnp.float32)
    HsT = jnp.tanh(S_b + jnp.dot(LmT, C_b, preferred_element_type=jnp.float32))
    HcT = jnp.tanh(C_b + jnp.dot(Lm, S_b, preferred_element_type=jnp.float32))

    vs = jnp.sum(HsT * whs_ref[...], axis=1, keepdims=True).reshape(TB, Ns, 1)
    As = _softmax_ax1(vs)
    As_ref[...] = As
    co_s = jnp.sum(XB3 * As, axis=1)

    vc = jnp.sum(HcT * whc_ref[...], axis=1, keepdims=True).reshape(TB, Nc, 1)
    Ac = _softmax_ax1(vc)
    Ac_ref[...] = Ac
    co_c = jnp.sum(XD3 * Ac, axis=1)

    cat = jnp.concatenate([xd_a, co_s, co_c], axis=1)
    logits = (jnp.dot(cat, Wfc_ref[...], preferred_element_type=jnp.float32)
              + bfc_ref[...])
    m = jnp.max(logits, axis=1, keepdims=True)
    e = jnp.exp(logits - m)
    preds_ref[...] = e / jnp.sum(e, axis=1, keepdims=True)


def _head_pass(xb, xd, W_att, b_att, u_att, Wl, Wc, Ws, whs, whc, Wfc, bfc):
    B, Ns, _ = xb.shape
    Nc = xd.shape[1]
    TB = 16
    while B % TB:
        TB //= 2

    W_att_p = jnp.pad(W_att, ((0, 0), (0, ATT_PAD - ATT_DIM)))
    b_att_p = jnp.pad(b_att, ((0, 0), (0, ATT_PAD - ATT_DIM)))
    u_row = jnp.pad(u_att.T, ((0, 0), (0, ATT_PAD - ATT_DIM)))
    Wc_p = jnp.pad(Wc, ((0, K_PAD - K_CO), (0, 0)))
    Ws_p = jnp.pad(Ws, ((0, K_PAD - K_CO), (0, 0)))
    whs_r = jnp.pad(whs, ((0, 0), (0, K_PAD - K_CO)))
    whc_r = jnp.pad(whc, ((0, 0), (0, K_PAD - K_CO)))

    def full(shape):
        return pl.BlockSpec(shape, lambda b, _n=len(shape): (0,) * _n)

    out_shape = (
        jax.ShapeDtypeStruct((B, NUM_CLASSES), jnp.float32),
        jax.ShapeDtypeStruct((B, Ns, 1), jnp.float32),
        jax.ShapeDtypeStruct((B, Nc, 1), jnp.float32),
        jax.ShapeDtypeStruct((B, Nc, 1), jnp.float32),
    )
    in_specs = [
        pl.BlockSpec((TB, Ns, LATENT), lambda b: (b, 0, 0)),
        pl.BlockSpec((TB, Nc, LATENT), lambda b: (b, 0, 0)),
        full((LATENT, ATT_PAD)),
        full((1, ATT_PAD)),
        full((1, ATT_PAD)),
        full((LATENT, LATENT)),
        full((K_PAD, LATENT)),
        full((K_PAD, LATENT)),
        full((1, K_PAD)),
        full((1, K_PAD)),
        full((3 * LATENT, NUM_CLASSES)),
        full((1, NUM_CLASSES)),
    ]
    out_specs = (
        pl.BlockSpec((TB, NUM_CLASSES), lambda b: (b, 0)),
        pl.BlockSpec((TB, Ns, 1), lambda b: (b, 0, 0)),
        pl.BlockSpec((TB, Nc, 1), lambda b: (b, 0, 0)),
        pl.BlockSpec((TB, Nc, 1), lambda b: (b, 0, 0)),
    )
    return pl.pallas_call(
        _head_kernel,
        out_shape=out_shape,
        grid=(B // TB,),
        in_specs=in_specs,
        out_specs=out_specs,
        compiler_params=pltpu.CompilerParams(dimension_semantics=("parallel",)),
    )(xb, xd, W_att_p, b_att_p, u_row, Wl, Wc_p, Ws_p, whs_r, whc_r, Wfc, bfc)


def kernel(content, comments, emo_emb, w1, b1, w2, b2, W_att, b_att, u_att,
           Wl, Wc, Ws, whs, whc, Wfc, bfc):
    B, NS, S = content.shape
    NC = comments.shape[1]

    wg = _build_group_weights(w1, w2)
    b_fused = jnp.concatenate([b1, b2], axis=1)

    xb = _conv_features(content.reshape(B * NS, S), emo_emb, wg,
                        b_fused).reshape(B, NS, LATENT)
    xd = _conv_features(comments.reshape(B * NC, S), emo_emb, wg,
                        b_fused).reshape(B, NC, LATENT)
    return _head_pass(xb, xd, W_att, b_att, u_att, Wl, Wc, Ws, whs, whc, Wfc, bfc)


# speedup vs baseline: 2.0760x; 1.0211x over previous
import jax
import jax.numpy as jnp
from jax.experimental import pallas as pl
from jax.experimental.pallas import tpu as pltpu

EMB = 32
SEQ = 16
CONV_OUT = 64
LATENT = 2 * CONV_OUT
ATT_DIM = 100
ATT_PAD = 128
K_CO = 80
K_PAD = 128
NUM_CLASSES = 2
EPS = 1e-7

TM = 2048
_CHUNK_ROWS = TM * SEQ // 128

_GROUPS = ((0, (0, 1, 2, 3, 4, 5)),
           (4, (6, 7, 8, 9)),
           (8, (10, 11, 12, 13, 14)))


def _round_up(a, b):
    return (a + b - 1) // b * b


def _conv_kernel(idx_ref, t0_ref, t1_ref, t2_ref, t3_ref,
                 wg0_ref, wg1_ref, wg2_ref, b_ref, out_ref,
                 x_scratch, idx_smem, sem):
    tbls = (t0_ref, t1_ref, t2_ref, t3_ref)

    cp = pltpu.make_async_copy(idx_ref, idx_smem, sem)
    cp.start()
    cp.wait()

    def chunk(c2, _):
        for dc in range(2):
            c = c2 * 2 + dc
            cols = []
            for g in range(4):
                rows = []
                for m in range(8):
                    k0 = m * SEQ + g * 4
                    p = (tbls[0][idx_smem[c, k0 + 0]]
                         + tbls[1][idx_smem[c, k0 + 1]])
                    q = (tbls[2][idx_smem[c, k0 + 2]]
                         + tbls[3][idx_smem[c, k0 + 3]])
                    rows.append(p + q)
                cols.append(jnp.concatenate(rows, axis=0))
            r0 = pl.multiple_of(c * 8, 8)
            for g in range(4):
                x_scratch[pl.ds(r0, 8), g * 128:(g + 1) * 128] = cols[g]
        return _

    jax.lax.fori_loop(0, _CHUNK_ROWS // 2, chunk, 0)

    x = x_scratch[...]
    ys = (
        jnp.dot(x[:, 0:256], wg0_ref[...], preferred_element_type=jnp.float32),
        jnp.dot(x[:, 128:384], wg1_ref[...], preferred_element_type=jnp.float32),
        jnp.dot(x[:, 256:512], wg2_ref[...], preferred_element_type=jnp.float32),
    )
    chunks = []
    for y, (_, ps) in zip(ys, _GROUPS):
        for i in range(len(ps)):
            chunks.append(y[:, i * LATENT:(i + 1) * LATENT])
    lane = jax.lax.broadcasted_iota(jnp.int32, chunks[-1].shape, 1)
    chunks[-1] = jnp.where(lane < CONV_OUT, chunks[-1], -jnp.inf)
    while len(chunks) > 1:
        nxt = [jnp.maximum(chunks[i], chunks[i + 1])
               for i in range(0, len(chunks) - 1, 2)]
        if len(chunks) % 2:
            nxt.append(chunks[-1])
        chunks = nxt
    out_ref[...] = jnp.maximum(chunks[0] + b_ref[...], 0.0)


def _build_group_weights(w1, w2):
    taps = [jnp.concatenate(
        [w1[t] if t < 2 else jnp.zeros((EMB, CONV_OUT), jnp.float32), w2[t]],
        axis=1) for t in range(3)]
    outs = []
    for base, ps in _GROUPS:
        W = jnp.zeros((8 * EMB, len(ps) * LATENT), jnp.float32)
        for i, p in enumerate(ps):
            for t in range(3):
                if p + t >= SEQ:
                    continue
                o = p - base + t
                W = W.at[o * EMB:(o + 1) * EMB, i * LATENT:(i + 1) * LATENT].set(taps[t])
        outs.append(W)
    return outs


def _conv_features(idx, emo_emb, wg, b_fused):
    N = idx.shape[0]
    N_pad = _round_up(N, TM)
    if N_pad != N:
        idx = jnp.pad(idx, ((0, N_pad - N), (0, 0)))
    idx2d = idx.reshape(N_pad * SEQ // 128, 128)

    V = emo_emb.shape[0]
    tbls = []
    for r in range(4):
        t = jnp.zeros((V, 128), jnp.float32).at[:, r * EMB:(r + 1) * EMB].set(emo_emb)
        tbls.append(t.reshape(V, 1, 128))

    out = pl.pallas_call(
        _conv_kernel,
        out_shape=jax.ShapeDtypeStruct((N_pad, LATENT), jnp.float32),
        grid=(N_pad // TM,),
        in_specs=[
            pl.BlockSpec((_CHUNK_ROWS, 128), lambda n: (n, 0)),
            pl.BlockSpec((V, 1, 128), lambda n: (0, 0, 0)),
            pl.BlockSpec((V, 1, 128), lambda n: (0, 0, 0)),
            pl.BlockSpec((V, 1, 128), lambda n: (0, 0, 0)),
            pl.BlockSpec((V, 1, 128), lambda n: (0, 0, 0)),
            pl.BlockSpec(wg[0].shape, lambda n: (0, 0)),
            pl.BlockSpec(wg[1].shape, lambda n: (0, 0)),
            pl.BlockSpec(wg[2].shape, lambda n: (0, 0)),
            pl.BlockSpec((1, LATENT), lambda n: (0, 0)),
        ],
        out_specs=pl.BlockSpec((TM, LATENT), lambda n: (n, 0)),
        scratch_shapes=[
            pltpu.VMEM((TM, SEQ * EMB), jnp.float32),
            pltpu.SMEM((_CHUNK_ROWS, 128), jnp.int32),
            pltpu.SemaphoreType.DMA,
        ],
        compiler_params=pltpu.CompilerParams(
            dimension_semantics=("parallel",),
            vmem_limit_bytes=56 * 1024 * 1024,
        ),
    )(idx2d, tbls[0], tbls[1], tbls[2], tbls[3], wg[0], wg[1], wg[2], b_fused)
    return out[:N]


def _softmax_ax1(x):
    m = jnp.max(x, axis=1, keepdims=True)
    e = jnp.exp(x - m)
    return e / jnp.sum(e, axis=1, keepdims=True)


def _head_kernel(xb_ref, xd_ref, Watt_ref, batt_ref, u_ref, Wl_ref, Wc_ref,
                 Ws_ref, whs_ref, whc_ref, Wfc_ref, bfc_ref,
                 preds_ref, As_ref, Ac_ref, ait_ref):
    TB, Ns, _ = xb_ref.shape
    Nc = xd_ref.shape[1]
    XB3 = xb_ref[...]
    XD3 = xd_ref[...]
    XB = XB3.reshape(TB * Ns, LATENT)
    XD = XD3.reshape(TB * Nc, LATENT)
    cn = (((1,), (1,)), ((), ()))

    uit = jnp.tanh(jnp.dot(XD, Watt_ref[...], preferred_element_type=jnp.float32)
                   + batt_ref[...])
    s = jnp.sum(uit.reshape(TB, Nc, LATENT) * u_ref[...].reshape(1, 1, LATENT),
                axis=2, keepdims=True)
    a = jnp.exp(s)
    an = a / (jnp.sum(a, axis=1, keepdims=True) + EPS)
    ait_ref[...] = an
    xd_a = jnp.sum(XD3 * an, axis=1)

    XDWl = jnp.dot(XD, Wl_ref[...], preferred_element_type=jnp.float32)
    Lbig = jax.lax.dot_general(XDWl, XB, cn,
                               preferred_element_type=jnp.float32)
    rb = jax.lax.broadcasted_iota(jnp.int32, Lbig.shape, 0) // Nc
    cb = jax.lax.broadcasted_iota(jnp.int32, Lbig.shape, 1) // Ns
    Lm = jnp.where(rb == cb, jnp.tanh(Lbig), 0.0)

    XBWlT = jax.lax.dot_general(XB, Wl_ref[...], cn,
                                preferred_element_type=jnp.float32)
    LbigT = jax.lax.dot_general(XBWlT, XD, cn,
                                preferred_element_type=jnp.float32)
    rbT = jax.lax.broadcasted_iota(jnp.int32, LbigT.shape, 0) // Ns
    cbT = jax.lax.broadcasted_iota(jnp.int32, LbigT.shape, 1) // Nc
    LmT = jnp.where(rbT == cbT, jnp.tanh(LbigT), 0.0)

    S_b = jax.lax.dot_general(XB, Ws_ref[...], cn,
                              preferred_element_type=jnp.float32)
    C_b = jax.lax.dot_general(XD, Wc_ref[...], cn,
                              preferred_element_type=jnp.float32)
    HsT = jnp.tanh(S_b + jnp.dot(LmT, C_b, preferred_element_type=jnp.float32))
    HcT = jnp.tanh(C_b + jnp.dot(Lm, S_b, preferred_element_type=jnp.float32))

    vs = jnp.sum(HsT * whs_ref[...], axis=1, keepdims=True).reshape(TB, Ns, 1)
    As = _softmax_ax1(vs)
    As_ref[...] = As
    co_s = jnp.sum(XB3 * As, axis=1)

    vc = jnp.sum(HcT * whc_ref[...], axis=1, keepdims=True).reshape(TB, Nc, 1)
    Ac = _softmax_ax1(vc)
    Ac_ref[...] = Ac
    co_c = jnp.sum(XD3 * Ac, axis=1)

    cat = jnp.concatenate([xd_a, co_s, co_c], axis=1)
    logits = (jnp.dot(cat, Wfc_ref[...], preferred_element_type=jnp.float32)
              + bfc_ref[...])
    m = jnp.max(logits, axis=1, keepdims=True)
    e = jnp.exp(logits - m)
    preds_ref[...] = e / jnp.sum(e, axis=1, keepdims=True)


def _head_pass(xb, xd, W_att, b_att, u_att, Wl, Wc, Ws, whs, whc, Wfc, bfc):
    B, Ns, _ = xb.shape
    Nc = xd.shape[1]
    TB = 16
    while B % TB:
        TB //= 2

    W_att_p = jnp.pad(W_att, ((0, 0), (0, ATT_PAD - ATT_DIM)))
    b_att_p = jnp.pad(b_att, ((0, 0), (0, ATT_PAD - ATT_DIM)))
    u_row = jnp.pad(u_att.T, ((0, 0), (0, ATT_PAD - ATT_DIM)))
    Wc_p = jnp.pad(Wc, ((0, K_PAD - K_CO), (0, 0)))
    Ws_p = jnp.pad(Ws, ((0, K_PAD - K_CO), (0, 0)))
    whs_r = jnp.pad(whs, ((0, 0), (0, K_PAD - K_CO)))
    whc_r = jnp.pad(whc, ((0, 0), (0, K_PAD - K_CO)))

    def full(shape):
        return pl.BlockSpec(shape, lambda b, _n=len(shape): (0,) * _n)

    out_shape = (
        jax.ShapeDtypeStruct((B, NUM_CLASSES), jnp.float32),
        jax.ShapeDtypeStruct((B, Ns, 1), jnp.float32),
        jax.ShapeDtypeStruct((B, Nc, 1), jnp.float32),
        jax.ShapeDtypeStruct((B, Nc, 1), jnp.float32),
    )
    in_specs = [
        pl.BlockSpec((TB, Ns, LATENT), lambda b: (b, 0, 0)),
        pl.BlockSpec((TB, Nc, LATENT), lambda b: (b, 0, 0)),
        full((LATENT, ATT_PAD)),
        full((1, ATT_PAD)),
        full((1, ATT_PAD)),
        full((LATENT, LATENT)),
        full((K_PAD, LATENT)),
        full((K_PAD, LATENT)),
        full((1, K_PAD)),
        full((1, K_PAD)),
        full((3 * LATENT, NUM_CLASSES)),
        full((1, NUM_CLASSES)),
    ]
    out_specs = (
        pl.BlockSpec((TB, NUM_CLASSES), lambda b: (b, 0)),
        pl.BlockSpec((TB, Ns, 1), lambda b: (b, 0, 0)),
        pl.BlockSpec((TB, Nc, 1), lambda b: (b, 0, 0)),
        pl.BlockSpec((TB, Nc, 1), lambda b: (b, 0, 0)),
    )
    return pl.pallas_call(
        _head_kernel,
        out_shape=out_shape,
        grid=(B // TB,),
        in_specs=in_specs,
        out_specs=out_specs,
        compiler_params=pltpu.CompilerParams(dimension_semantics=("parallel",)),
    )(xb, xd, W_att_p, b_att_p, u_row, Wl, Wc_p, Ws_p, whs_r, whc_r, Wfc, bfc)


def kernel(content, comments, emo_emb, w1, b1, w2, b2, W_att, b_att, u_att,
           Wl, Wc, Ws, whs, whc, Wfc, bfc):
    B, NS, S = content.shape
    NC = comments.shape[1]

    wg = _build_group_weights(w1, w2)
    b_fused = jnp.concatenate([b1, b2], axis=1)

    xb = _conv_features(content.reshape(B * NS, S), emo_emb, wg,
                        b_fused).reshape(B, NS, LATENT)
    xd = _conv_features(comments.reshape(B * NC, S), emo_emb, wg,
                        b_fused).reshape(B, NC, LATENT)
    return _head_pass(xb, xd, W_att, b_att, u_att, Wl, Wc, Ws, whs, whc, Wfc, bfc)


# speedup vs baseline: 2.1747x; 1.0475x over previous
import jax
import jax.numpy as jnp
from jax.experimental import pallas as pl
from jax.experimental.pallas import tpu as pltpu

EMB = 32
SEQ = 16
CONV_OUT = 64
LATENT = 2 * CONV_OUT
ATT_DIM = 100
ATT_PAD = 128
K_CO = 80
K_PAD = 128
NUM_CLASSES = 2
EPS = 1e-7

TM = 2048
_CHUNK_ROWS = TM * SEQ // 128

_GROUPS = ((0, (0, 1, 2, 3, 4, 5)),
           (4, (6, 7, 8, 9)),
           (8, (10, 11, 12, 13, 14)))


def _round_up(a, b):
    return (a + b - 1) // b * b


def _conv_kernel(idx_ref, t0_ref, t1_ref, t2_ref, t3_ref,
                 wg0_ref, wg1_ref, wg2_ref, b_ref, out_ref,
                 x_scratch, idx_smem, sem):
    tbls = (t0_ref, t1_ref, t2_ref, t3_ref)

    QR = _CHUNK_ROWS // 4
    cps = []
    for qtr in range(4):
        cp = pltpu.make_async_copy(idx_ref.at[pl.ds(qtr * QR, QR), :],
                                   idx_smem.at[pl.ds(qtr * QR, QR), :],
                                   sem.at[qtr])
        cp.start()
        cps.append(cp)

    def chunk(c2, _):
        for dc in range(2):
            c = c2 * 2 + dc
            cols = []
            for g in range(4):
                rows = []
                for m in range(8):
                    k0 = m * SEQ + g * 4
                    p = (tbls[0][idx_smem[c, k0 + 0]]
                         + tbls[1][idx_smem[c, k0 + 1]])
                    q = (tbls[2][idx_smem[c, k0 + 2]]
                         + tbls[3][idx_smem[c, k0 + 3]])
                    rows.append(p + q)
                cols.append(jnp.concatenate(rows, axis=0))
            r0 = pl.multiple_of(c * 8, 8)
            for g in range(4):
                x_scratch[pl.ds(r0, 8), g * 128:(g + 1) * 128] = cols[g]
        return _

    for qtr in range(4):
        cps[qtr].wait()
        jax.lax.fori_loop(qtr * QR // 2, (qtr + 1) * QR // 2, chunk, 0)

    x = x_scratch[...]
    ys = (
        jnp.dot(x[:, 0:256], wg0_ref[...], preferred_element_type=jnp.float32),
        jnp.dot(x[:, 128:384], wg1_ref[...], preferred_element_type=jnp.float32),
        jnp.dot(x[:, 256:512], wg2_ref[...], preferred_element_type=jnp.float32),
    )
    chunks = []
    for y, (_, ps) in zip(ys, _GROUPS):
        for i in range(len(ps)):
            chunks.append(y[:, i * LATENT:(i + 1) * LATENT])
    lane = jax.lax.broadcasted_iota(jnp.int32, chunks[-1].shape, 1)
    chunks[-1] = jnp.where(lane < CONV_OUT, chunks[-1], -jnp.inf)
    while len(chunks) > 1:
        nxt = [jnp.maximum(chunks[i], chunks[i + 1])
               for i in range(0, len(chunks) - 1, 2)]
        if len(chunks) % 2:
            nxt.append(chunks[-1])
        chunks = nxt
    out_ref[...] = jnp.maximum(chunks[0] + b_ref[...], 0.0)


def _build_group_weights(w1, w2):
    taps = [jnp.concatenate(
        [w1[t] if t < 2 else jnp.zeros((EMB, CONV_OUT), jnp.float32), w2[t]],
        axis=1) for t in range(3)]
    outs = []
    for base, ps in _GROUPS:
        W = jnp.zeros((8 * EMB, len(ps) * LATENT), jnp.float32)
        for i, p in enumerate(ps):
            for t in range(3):
                if p + t >= SEQ:
                    continue
                o = p - base + t
                W = W.at[o * EMB:(o + 1) * EMB, i * LATENT:(i + 1) * LATENT].set(taps[t])
        outs.append(W)
    return outs


def _conv_features(idx, emo_emb, wg, b_fused):
    N = idx.shape[0]
    N_pad = _round_up(N, TM)
    if N_pad != N:
        idx = jnp.pad(idx, ((0, N_pad - N), (0, 0)))
    idx2d = idx.reshape(N_pad * SEQ // 128, 128)

    V = emo_emb.shape[0]
    tbls = []
    for r in range(4):
        t = jnp.zeros((V, 128), jnp.float32).at[:, r * EMB:(r + 1) * EMB].set(emo_emb)
        tbls.append(t.reshape(V, 1, 128))

    out = pl.pallas_call(
        _conv_kernel,
        out_shape=jax.ShapeDtypeStruct((N_pad, LATENT), jnp.float32),
        grid=(N_pad // TM,),
        in_specs=[
            pl.BlockSpec((_CHUNK_ROWS, 128), lambda n: (n, 0)),
            pl.BlockSpec((V, 1, 128), lambda n: (0, 0, 0)),
            pl.BlockSpec((V, 1, 128), lambda n: (0, 0, 0)),
            pl.BlockSpec((V, 1, 128), lambda n: (0, 0, 0)),
            pl.BlockSpec((V, 1, 128), lambda n: (0, 0, 0)),
            pl.BlockSpec(wg[0].shape, lambda n: (0, 0)),
            pl.BlockSpec(wg[1].shape, lambda n: (0, 0)),
            pl.BlockSpec(wg[2].shape, lambda n: (0, 0)),
            pl.BlockSpec((1, LATENT), lambda n: (0, 0)),
        ],
        out_specs=pl.BlockSpec((TM, LATENT), lambda n: (n, 0)),
        scratch_shapes=[
            pltpu.VMEM((TM, SEQ * EMB), jnp.float32),
            pltpu.SMEM((_CHUNK_ROWS, 128), jnp.int32),
            pltpu.SemaphoreType.DMA((4,)),
        ],
        compiler_params=pltpu.CompilerParams(
            dimension_semantics=("parallel",),
            vmem_limit_bytes=56 * 1024 * 1024,
        ),
    )(idx2d, tbls[0], tbls[1], tbls[2], tbls[3], wg[0], wg[1], wg[2], b_fused)
    return out[:N]


def _softmax_ax1(x):
    m = jnp.max(x, axis=1, keepdims=True)
    e = jnp.exp(x - m)
    return e / jnp.sum(e, axis=1, keepdims=True)


def _head_kernel(xb_ref, xd_ref, Watt_ref, batt_ref, u_ref, Wl_ref, Wc_ref,
                 Ws_ref, whs_ref, whc_ref, Wfc_ref, bfc_ref,
                 preds_ref, As_ref, Ac_ref, ait_ref):
    TB, Ns, _ = xb_ref.shape
    Nc = xd_ref.shape[1]
    XB3 = xb_ref[...]
    XD3 = xd_ref[...]
    XB = XB3.reshape(TB * Ns, LATENT)
    XD = XD3.reshape(TB * Nc, LATENT)
    cn = (((1,), (1,)), ((), ()))

    uit = jnp.tanh(jnp.dot(XD, Watt_ref[...], preferred_element_type=jnp.float32)
                   + batt_ref[...])
    s = jnp.sum(uit.reshape(TB, Nc, LATENT) * u_ref[...].reshape(1, 1, LATENT),
                axis=2, keepdims=True)
    a = jnp.exp(s)
    an = a / (jnp.sum(a, axis=1, keepdims=True) + EPS)
    ait_ref[...] = an
    xd_a = jnp.sum(XD3 * an, axis=1)

    XDWl = jnp.dot(XD, Wl_ref[...], preferred_element_type=jnp.float32)
    Lbig = jax.lax.dot_general(XDWl, XB, cn,
                               preferred_element_type=jnp.float32)
    rb = jax.lax.broadcasted_iota(jnp.int32, Lbig.shape, 0) // Nc
    cb = jax.lax.broadcasted_iota(jnp.int32, Lbig.shape, 1) // Ns
    Lm = jnp.where(rb == cb, jnp.tanh(Lbig), 0.0)

    XBWlT = jax.lax.dot_general(XB, Wl_ref[...], cn,
                                preferred_element_type=jnp.float32)
    LbigT = jax.lax.dot_general(XBWlT, XD, cn,
                                preferred_element_type=jnp.float32)
    rbT = jax.lax.broadcasted_iota(jnp.int32, LbigT.shape, 0) // Ns
    cbT = jax.lax.broadcasted_iota(jnp.int32, LbigT.shape, 1) // Nc
    LmT = jnp.where(rbT == cbT, jnp.tanh(LbigT), 0.0)

    S_b = jax.lax.dot_general(XB, Ws_ref[...], cn,
                              preferred_element_type=jnp.float32)
    C_b = jax.lax.dot_general(XD, Wc_ref[...], cn,
                              preferred_element_type=jnp.float32)
    HsT = jnp.tanh(S_b + jnp.dot(LmT, C_b, preferred_element_type=jnp.float32))
    HcT = jnp.tanh(C_b + jnp.dot(Lm, S_b, preferred_element_type=jnp.float32))

    vs = jnp.sum(HsT * whs_ref[...], axis=1, keepdims=True).reshape(TB, Ns, 1)
    As = _softmax_ax1(vs)
    As_ref[...] = As
    co_s = jnp.sum(XB3 * As, axis=1)

    vc = jnp.sum(HcT * whc_ref[...], axis=1, keepdims=True).reshape(TB, Nc, 1)
    Ac = _softmax_ax1(vc)
    Ac_ref[...] = Ac
    co_c = jnp.sum(XD3 * Ac, axis=1)

    cat = jnp.concatenate([xd_a, co_s, co_c], axis=1)
    logits = (jnp.dot(cat, Wfc_ref[...], preferred_element_type=jnp.float32)
              + bfc_ref[...])
    m = jnp.max(logits, axis=1, keepdims=True)
    e = jnp.exp(logits - m)
    preds_ref[...] = e / jnp.sum(e, axis=1, keepdims=True)


def _head_pass(xb, xd, W_att, b_att, u_att, Wl, Wc, Ws, whs, whc, Wfc, bfc):
    B, Ns, _ = xb.shape
    Nc = xd.shape[1]
    TB = 16
    while B % TB:
        TB //= 2

    W_att_p = jnp.pad(W_att, ((0, 0), (0, ATT_PAD - ATT_DIM)))
    b_att_p = jnp.pad(b_att, ((0, 0), (0, ATT_PAD - ATT_DIM)))
    u_row = jnp.pad(u_att.T, ((0, 0), (0, ATT_PAD - ATT_DIM)))
    Wc_p = jnp.pad(Wc, ((0, K_PAD - K_CO), (0, 0)))
    Ws_p = jnp.pad(Ws, ((0, K_PAD - K_CO), (0, 0)))
    whs_r = jnp.pad(whs, ((0, 0), (0, K_PAD - K_CO)))
    whc_r = jnp.pad(whc, ((0, 0), (0, K_PAD - K_CO)))

    def full(shape):
        return pl.BlockSpec(shape, lambda b, _n=len(shape): (0,) * _n)

    out_shape = (
        jax.ShapeDtypeStruct((B, NUM_CLASSES), jnp.float32),
        jax.ShapeDtypeStruct((B, Ns, 1), jnp.float32),
        jax.ShapeDtypeStruct((B, Nc, 1), jnp.float32),
        jax.ShapeDtypeStruct((B, Nc, 1), jnp.float32),
    )
    in_specs = [
        pl.BlockSpec((TB, Ns, LATENT), lambda b: (b, 0, 0)),
        pl.BlockSpec((TB, Nc, LATENT), lambda b: (b, 0, 0)),
        full((LATENT, ATT_PAD)),
        full((1, ATT_PAD)),
        full((1, ATT_PAD)),
        full((LATENT, LATENT)),
        full((K_PAD, LATENT)),
        full((K_PAD, LATENT)),
        full((1, K_PAD)),
        full((1, K_PAD)),
        full((3 * LATENT, NUM_CLASSES)),
        full((1, NUM_CLASSES)),
    ]
    out_specs = (
        pl.BlockSpec((TB, NUM_CLASSES), lambda b: (b, 0)),
        pl.BlockSpec((TB, Ns, 1), lambda b: (b, 0, 0)),
        pl.BlockSpec((TB, Nc, 1), lambda b: (b, 0, 0)),
        pl.BlockSpec((TB, Nc, 1), lambda b: (b, 0, 0)),
    )
    return pl.pallas_call(
        _head_kernel,
        out_shape=out_shape,
        grid=(B // TB,),
        in_specs=in_specs,
        out_specs=out_specs,
        compiler_params=pltpu.CompilerParams(dimension_semantics=("parallel",)),
    )(xb, xd, W_att_p, b_att_p, u_row, Wl, Wc_p, Ws_p, whs_r, whc_r, Wfc, bfc)


def kernel(content, comments, emo_emb, w1, b1, w2, b2, W_att, b_att, u_att,
           Wl, Wc, Ws, whs, whc, Wfc, bfc):
    B, NS, S = content.shape
    NC = comments.shape[1]

    wg = _build_group_weights(w1, w2)
    b_fused = jnp.concatenate([b1, b2], axis=1)

    xb = _conv_features(content.reshape(B * NS, S), emo_emb, wg,
                        b_fused).reshape(B, NS, LATENT)
    xd = _conv_features(comments.reshape(B * NC, S), emo_emb, wg,
                        b_fused).reshape(B, NC, LATENT)
    return _head_pass(xb, xd, W_att, b_att, u_att, Wl, Wc, Ws, whs, whc, Wfc, bfc)


# speedup vs baseline: 2.2051x; 1.0140x over previous
import jax
import jax.numpy as jnp
from jax.experimental import pallas as pl
from jax.experimental.pallas import tpu as pltpu

EMB = 32
SEQ = 16
CONV_OUT = 64
LATENT = 2 * CONV_OUT
ATT_DIM = 100
ATT_PAD = 128
K_CO = 80
K_PAD = 128
NUM_CLASSES = 2
EPS = 1e-7

TM = 2048
_CHUNK_ROWS = TM * SEQ // 128

_GROUPS = ((0, (0, 1, 2, 3, 4, 5)),
           (4, (6, 7, 8, 9)),
           (8, (10, 11, 12, 13, 14)))


def _round_up(a, b):
    return (a + b - 1) // b * b


def _conv_kernel(idx_ref, t0_ref, t1_ref, t2_ref, t3_ref,
                 wg0_ref, wg1_ref, wg2_ref, b_ref, out_ref,
                 x_scratch, idx_smem, sem):
    tbls = (t0_ref, t1_ref, t2_ref, t3_ref)

    QR = _CHUNK_ROWS // 4
    cps = []
    for qtr in range(4):
        cp = pltpu.make_async_copy(idx_ref.at[pl.ds(qtr * QR, QR), :],
                                   idx_smem.at[pl.ds(qtr * QR, QR), :],
                                   sem.at[qtr])
        cp.start()
        cps.append(cp)

    def chunk(c2, _):
        for dc in range(8):
            c = c2 * 8 + dc
            cols = []
            for g in range(4):
                rows = []
                for m in range(8):
                    k0 = m * SEQ + g * 4
                    p = (tbls[0][idx_smem[c, k0 + 0]]
                         + tbls[1][idx_smem[c, k0 + 1]])
                    q = (tbls[2][idx_smem[c, k0 + 2]]
                         + tbls[3][idx_smem[c, k0 + 3]])
                    rows.append(p + q)
                cols.append(jnp.concatenate(rows, axis=0))
            r0 = pl.multiple_of(c * 8, 8)
            for g in range(4):
                x_scratch[pl.ds(r0, 8), g * 128:(g + 1) * 128] = cols[g]
        return _

    for qtr in range(4):
        cps[qtr].wait()
        jax.lax.fori_loop(qtr * QR // 8, (qtr + 1) * QR // 8, chunk, 0)

    x = x_scratch[...]
    ys = (
        jnp.dot(x[:, 0:256], wg0_ref[...], preferred_element_type=jnp.float32),
        jnp.dot(x[:, 128:384], wg1_ref[...], preferred_element_type=jnp.float32),
        jnp.dot(x[:, 256:512], wg2_ref[...], preferred_element_type=jnp.float32),
    )
    chunks = []
    for y, (_, ps) in zip(ys, _GROUPS):
        for i in range(len(ps)):
            chunks.append(y[:, i * LATENT:(i + 1) * LATENT])
    lane = jax.lax.broadcasted_iota(jnp.int32, chunks[-1].shape, 1)
    chunks[-1] = jnp.where(lane < CONV_OUT, chunks[-1], -jnp.inf)
    while len(chunks) > 1:
        nxt = [jnp.maximum(chunks[i], chunks[i + 1])
               for i in range(0, len(chunks) - 1, 2)]
        if len(chunks) % 2:
            nxt.append(chunks[-1])
        chunks = nxt
    out_ref[...] = jnp.maximum(chunks[0] + b_ref[...], 0.0)


def _build_group_weights(w1, w2):
    taps = [jnp.concatenate(
        [w1[t] if t < 2 else jnp.zeros((EMB, CONV_OUT), jnp.float32), w2[t]],
        axis=1) for t in range(3)]
    outs = []
    for base, ps in _GROUPS:
        W = jnp.zeros((8 * EMB, len(ps) * LATENT), jnp.float32)
        for i, p in enumerate(ps):
            for t in range(3):
                if p + t >= SEQ:
                    continue
                o = p - base + t
                W = W.at[o * EMB:(o + 1) * EMB, i * LATENT:(i + 1) * LATENT].set(taps[t])
        outs.append(W)
    return outs


def _conv_features(idx, emo_emb, wg, b_fused):
    N = idx.shape[0]
    N_pad = _round_up(N, TM)
    if N_pad != N:
        idx = jnp.pad(idx, ((0, N_pad - N), (0, 0)))
    idx2d = idx.reshape(N_pad * SEQ // 128, 128)

    V = emo_emb.shape[0]
    tbls = []
    for r in range(4):
        t = jnp.zeros((V, 128), jnp.float32).at[:, r * EMB:(r + 1) * EMB].set(emo_emb)
        tbls.append(t.reshape(V, 1, 128))

    out = pl.pallas_call(
        _conv_kernel,
        out_shape=jax.ShapeDtypeStruct((N_pad, LATENT), jnp.float32),
        grid=(N_pad // TM,),
        in_specs=[
            pl.BlockSpec((_CHUNK_ROWS, 128), lambda n: (n, 0)),
            pl.BlockSpec((V, 1, 128), lambda n: (0, 0, 0)),
            pl.BlockSpec((V, 1, 128), lambda n: (0, 0, 0)),
            pl.BlockSpec((V, 1, 128), lambda n: (0, 0, 0)),
            pl.BlockSpec((V, 1, 128), lambda n: (0, 0, 0)),
            pl.BlockSpec(wg[0].shape, lambda n: (0, 0)),
            pl.BlockSpec(wg[1].shape, lambda n: (0, 0)),
            pl.BlockSpec(wg[2].shape, lambda n: (0, 0)),
            pl.BlockSpec((1, LATENT), lambda n: (0, 0)),
        ],
        out_specs=pl.BlockSpec((TM, LATENT), lambda n: (n, 0)),
        scratch_shapes=[
            pltpu.VMEM((TM, SEQ * EMB), jnp.float32),
            pltpu.SMEM((_CHUNK_ROWS, 128), jnp.int32),
            pltpu.SemaphoreType.DMA((4,)),
        ],
        compiler_params=pltpu.CompilerParams(
            dimension_semantics=("parallel",),
            vmem_limit_bytes=56 * 1024 * 1024,
        ),
    )(idx2d, tbls[0], tbls[1], tbls[2], tbls[3], wg[0], wg[1], wg[2], b_fused)
    return out[:N]


def _softmax_ax1(x):
    m = jnp.max(x, axis=1, keepdims=True)
    e = jnp.exp(x - m)
    return e / jnp.sum(e, axis=1, keepdims=True)


def _head_kernel(xb_ref, xd_ref, Watt_ref, batt_ref, u_ref, Wl_ref, Wc_ref,
                 Ws_ref, whs_ref, whc_ref, Wfc_ref, bfc_ref,
                 preds_ref, As_ref, Ac_ref, ait_ref):
    TB, Ns, _ = xb_ref.shape
    Nc = xd_ref.shape[1]
    XB3 = xb_ref[...]
    XD3 = xd_ref[...]
    XB = XB3.reshape(TB * Ns, LATENT)
    XD = XD3.reshape(TB * Nc, LATENT)
    cn = (((1,), (1,)), ((), ()))

    uit = jnp.tanh(jnp.dot(XD, Watt_ref[...], preferred_element_type=jnp.float32)
                   + batt_ref[...])
    s = jnp.sum(uit.reshape(TB, Nc, LATENT) * u_ref[...].reshape(1, 1, LATENT),
                axis=2, keepdims=True)
    a = jnp.exp(s)
    an = a / (jnp.sum(a, axis=1, keepdims=True) + EPS)
    ait_ref[...] = an
    xd_a = jnp.sum(XD3 * an, axis=1)

    XDWl = jnp.dot(XD, Wl_ref[...], preferred_element_type=jnp.float32)
    Lbig = jax.lax.dot_general(XDWl, XB, cn,
                               preferred_element_type=jnp.float32)
    rb = jax.lax.broadcasted_iota(jnp.int32, Lbig.shape, 0) // Nc
    cb = jax.lax.broadcasted_iota(jnp.int32, Lbig.shape, 1) // Ns
    Lm = jnp.where(rb == cb, jnp.tanh(Lbig), 0.0)

    XBWlT = jax.lax.dot_general(XB, Wl_ref[...], cn,
                                preferred_element_type=jnp.float32)
    LbigT = jax.lax.dot_general(XBWlT, XD, cn,
                                preferred_element_type=jnp.float32)
    rbT = jax.lax.broadcasted_iota(jnp.int32, LbigT.shape, 0) // Ns
    cbT = jax.lax.broadcasted_iota(jnp.int32, LbigT.shape, 1) // Nc
    LmT = jnp.where(rbT == cbT, jnp.tanh(LbigT), 0.0)

    S_b = jax.lax.dot_general(XB, Ws_ref[...], cn,
                              preferred_element_type=jnp.float32)
    C_b = jax.lax.dot_general(XD, Wc_ref[...], cn,
                              preferred_element_type=jnp.float32)
    HsT = jnp.tanh(S_b + jnp.dot(LmT, C_b, preferred_element_type=jnp.float32))
    HcT = jnp.tanh(C_b + jnp.dot(Lm, S_b, preferred_element_type=jnp.float32))

    vs = jnp.sum(HsT * whs_ref[...], axis=1, keepdims=True).reshape(TB, Ns, 1)
    As = _softmax_ax1(vs)
    As_ref[...] = As
    co_s = jnp.sum(XB3 * As, axis=1)

    vc = jnp.sum(HcT * whc_ref[...], axis=1, keepdims=True).reshape(TB, Nc, 1)
    Ac = _softmax_ax1(vc)
    Ac_ref[...] = Ac
    co_c = jnp.sum(XD3 * Ac, axis=1)

    cat = jnp.concatenate([xd_a, co_s, co_c], axis=1)
    logits = (jnp.dot(cat, Wfc_ref[...], preferred_element_type=jnp.float32)
              + bfc_ref[...])
    m = jnp.max(logits, axis=1, keepdims=True)
    e = jnp.exp(logits - m)
    preds_ref[...] = e / jnp.sum(e, axis=1, keepdims=True)


def _head_pass(xb, xd, W_att, b_att, u_att, Wl, Wc, Ws, whs, whc, Wfc, bfc):
    B, Ns, _ = xb.shape
    Nc = xd.shape[1]
    TB = 16
    while B % TB:
        TB //= 2

    W_att_p = jnp.pad(W_att, ((0, 0), (0, ATT_PAD - ATT_DIM)))
    b_att_p = jnp.pad(b_att, ((0, 0), (0, ATT_PAD - ATT_DIM)))
    u_row = jnp.pad(u_att.T, ((0, 0), (0, ATT_PAD - ATT_DIM)))
    Wc_p = jnp.pad(Wc, ((0, K_PAD - K_CO), (0, 0)))
    Ws_p = jnp.pad(Ws, ((0, K_PAD - K_CO), (0, 0)))
    whs_r = jnp.pad(whs, ((0, 0), (0, K_PAD - K_CO)))
    whc_r = jnp.pad(whc, ((0, 0), (0, K_PAD - K_CO)))

    def full(shape):
        return pl.BlockSpec(shape, lambda b, _n=len(shape): (0,) * _n)

    out_shape = (
        jax.ShapeDtypeStruct((B, NUM_CLASSES), jnp.float32),
        jax.ShapeDtypeStruct((B, Ns, 1), jnp.float32),
        jax.ShapeDtypeStruct((B, Nc, 1), jnp.float32),
        jax.ShapeDtypeStruct((B, Nc, 1), jnp.float32),
    )
    in_specs = [
        pl.BlockSpec((TB, Ns, LATENT), lambda b: (b, 0, 0)),
        pl.BlockSpec((TB, Nc, LATENT), lambda b: (b, 0, 0)),
        full((LATENT, ATT_PAD)),
        full((1, ATT_PAD)),
        full((1, ATT_PAD)),
        full((LATENT, LATENT)),
        full((K_PAD, LATENT)),
        full((K_PAD, LATENT)),
        full((1, K_PAD)),
        full((1, K_PAD)),
        full((3 * LATENT, NUM_CLASSES)),
        full((1, NUM_CLASSES)),
    ]
    out_specs = (
        pl.BlockSpec((TB, NUM_CLASSES), lambda b: (b, 0)),
        pl.BlockSpec((TB, Ns, 1), lambda b: (b, 0, 0)),
        pl.BlockSpec((TB, Nc, 1), lambda b: (b, 0, 0)),
        pl.BlockSpec((TB, Nc, 1), lambda b: (b, 0, 0)),
    )
    return pl.pallas_call(
        _head_kernel,
        out_shape=out_shape,
        grid=(B // TB,),
        in_specs=in_specs,
        out_specs=out_specs,
        compiler_params=pltpu.CompilerParams(dimension_semantics=("parallel",)),
    )(xb, xd, W_att_p, b_att_p, u_row, Wl, Wc_p, Ws_p, whs_r, whc_r, Wfc, bfc)


def kernel(content, comments, emo_emb, w1, b1, w2, b2, W_att, b_att, u_att,
           Wl, Wc, Ws, whs, whc, Wfc, bfc):
    B, NS, S = content.shape
    NC = comments.shape[1]

    wg = _build_group_weights(w1, w2)
    b_fused = jnp.concatenate([b1, b2], axis=1)

    xb = _conv_features(content.reshape(B * NS, S), emo_emb, wg,
                        b_fused).reshape(B, NS, LATENT)
    xd = _conv_features(comments.reshape(B * NC, S), emo_emb, wg,
                        b_fused).reshape(B, NC, LATENT)
    return _head_pass(xb, xd, W_att, b_att, u_att, Wl, Wc, Ws, whs, whc, Wfc, bfc)


# speedup vs baseline: 2.7269x; 1.2366x over previous
import jax
import jax.numpy as jnp
from jax.experimental import pallas as pl
from jax.experimental.pallas import tpu as pltpu

EMB = 32
SEQ = 16
CONV_OUT = 64
LATENT = 2 * CONV_OUT
ATT_DIM = 100
ATT_PAD = 128
K_CO = 80
K_PAD = 128
NUM_CLASSES = 2
EPS = 1e-7

TM = 512
_CHUNK_ROWS = TM * SEQ // 128

_GROUPS = ((0, (0, 1, 2, 3, 4, 5)),
           (4, (6, 7, 8, 9)),
           (8, (10, 11, 12, 13, 14)))


def _round_up(a, b):
    return (a + b - 1) // b * b


def _conv_kernel(idx_ref, t0_ref, t1_ref, t2_ref, t3_ref,
                 wg0_ref, wg1_ref, wg2_ref, b_ref, out_ref,
                 x_scratch, idx_smem, sem):
    tbls = (t0_ref, t1_ref, t2_ref, t3_ref)

    QR = _CHUNK_ROWS // 4
    cps = []
    for qtr in range(4):
        cp = pltpu.make_async_copy(idx_ref.at[pl.ds(qtr * QR, QR), :],
                                   idx_smem.at[pl.ds(qtr * QR, QR), :],
                                   sem.at[qtr])
        cp.start()
        cps.append(cp)

    for qtr in range(4):
        cps[qtr].wait()
        for c in range(qtr * QR, (qtr + 1) * QR):
            for m in range(8):
                for g in range(4):
                    k0 = m * SEQ + g * 4
                    p = (tbls[0][idx_smem[c, k0 + 0]]
                         + tbls[1][idx_smem[c, k0 + 1]])
                    q = (tbls[2][idx_smem[c, k0 + 2]]
                         + tbls[3][idx_smem[c, k0 + 3]])
                    r = c * 8 + m
                    x_scratch[r:r + 1, g * 128:(g + 1) * 128] = p + q

    x = x_scratch[...]
    ys = (
        jnp.dot(x[:, 0:256], wg0_ref[...], preferred_element_type=jnp.float32),
        jnp.dot(x[:, 128:384], wg1_ref[...], preferred_element_type=jnp.float32),
        jnp.dot(x[:, 256:512], wg2_ref[...], preferred_element_type=jnp.float32),
    )
    chunks = []
    for y, (_, ps) in zip(ys, _GROUPS):
        for i in range(len(ps)):
            chunks.append(y[:, i * LATENT:(i + 1) * LATENT])
    lane = jax.lax.broadcasted_iota(jnp.int32, chunks[-1].shape, 1)
    chunks[-1] = jnp.where(lane < CONV_OUT, chunks[-1], -jnp.inf)
    while len(chunks) > 1:
        nxt = [jnp.maximum(chunks[i], chunks[i + 1])
               for i in range(0, len(chunks) - 1, 2)]
        if len(chunks) % 2:
            nxt.append(chunks[-1])
        chunks = nxt
    out_ref[...] = jnp.maximum(chunks[0] + b_ref[...], 0.0)


def _build_group_weights(w1, w2):
    taps = [jnp.concatenate(
        [w1[t] if t < 2 else jnp.zeros((EMB, CONV_OUT), jnp.float32), w2[t]],
        axis=1) for t in range(3)]
    outs = []
    for base, ps in _GROUPS:
        W = jnp.zeros((8 * EMB, len(ps) * LATENT), jnp.float32)
        for i, p in enumerate(ps):
            for t in range(3):
                if p + t >= SEQ:
                    continue
                o = p - base + t
                W = W.at[o * EMB:(o + 1) * EMB, i * LATENT:(i + 1) * LATENT].set(taps[t])
        outs.append(W)
    return outs


def _conv_features(idx, emo_emb, wg, b_fused):
    N = idx.shape[0]
    N_pad = _round_up(N, TM)
    if N_pad != N:
        idx = jnp.pad(idx, ((0, N_pad - N), (0, 0)))
    idx2d = idx.reshape(N_pad * SEQ // 128, 128)

    V = emo_emb.shape[0]
    tbls = []
    for r in range(4):
        t = jnp.zeros((V, 128), jnp.float32).at[:, r * EMB:(r + 1) * EMB].set(emo_emb)
        tbls.append(t.reshape(V, 1, 128))

    out = pl.pallas_call(
        _conv_kernel,
        out_shape=jax.ShapeDtypeStruct((N_pad, LATENT), jnp.float32),
        grid=(N_pad // TM,),
        in_specs=[
            pl.BlockSpec((_CHUNK_ROWS, 128), lambda n: (n, 0)),
            pl.BlockSpec((V, 1, 128), lambda n: (0, 0, 0)),
            pl.BlockSpec((V, 1, 128), lambda n: (0, 0, 0)),
            pl.BlockSpec((V, 1, 128), lambda n: (0, 0, 0)),
            pl.BlockSpec((V, 1, 128), lambda n: (0, 0, 0)),
            pl.BlockSpec(wg[0].shape, lambda n: (0, 0)),
            pl.BlockSpec(wg[1].shape, lambda n: (0, 0)),
            pl.BlockSpec(wg[2].shape, lambda n: (0, 0)),
            pl.BlockSpec((1, LATENT), lambda n: (0, 0)),
        ],
        out_specs=pl.BlockSpec((TM, LATENT), lambda n: (n, 0)),
        scratch_shapes=[
            pltpu.VMEM((TM, SEQ * EMB), jnp.float32),
            pltpu.SMEM((_CHUNK_ROWS, 128), jnp.int32),
            pltpu.SemaphoreType.DMA((4,)),
        ],
        compiler_params=pltpu.CompilerParams(
            dimension_semantics=("parallel",),
            vmem_limit_bytes=56 * 1024 * 1024,
        ),
    )(idx2d, tbls[0], tbls[1], tbls[2], tbls[3], wg[0], wg[1], wg[2], b_fused)
    return out[:N]


def _softmax_ax1(x):
    m = jnp.max(x, axis=1, keepdims=True)
    e = jnp.exp(x - m)
    return e / jnp.sum(e, axis=1, keepdims=True)


def _head_kernel(xb_ref, xd_ref, Watt_ref, batt_ref, u_ref, Wl_ref, Wc_ref,
                 Ws_ref, whs_ref, whc_ref, Wfc_ref, bfc_ref,
                 preds_ref, As_ref, Ac_ref, ait_ref):
    TB, Ns, _ = xb_ref.shape
    Nc = xd_ref.shape[1]
    XB3 = xb_ref[...]
    XD3 = xd_ref[...]
    XB = XB3.reshape(TB * Ns, LATENT)
    XD = XD3.reshape(TB * Nc, LATENT)
    cn = (((1,), (1,)), ((), ()))

    uit = jnp.tanh(jnp.dot(XD, Watt_ref[...], preferred_element_type=jnp.float32)
                   + batt_ref[...])
    s = jnp.sum(uit.reshape(TB, Nc, LATENT) * u_ref[...].reshape(1, 1, LATENT),
                axis=2, keepdims=True)
    a = jnp.exp(s)
    an = a / (jnp.sum(a, axis=1, keepdims=True) + EPS)
    ait_ref[...] = an
    xd_a = jnp.sum(XD3 * an, axis=1)

    XDWl = jnp.dot(XD, Wl_ref[...], preferred_element_type=jnp.float32)
    Lbig = jax.lax.dot_general(XDWl, XB, cn,
                               preferred_element_type=jnp.float32)
    rb = jax.lax.broadcasted_iota(jnp.int32, Lbig.shape, 0) // Nc
    cb = jax.lax.broadcasted_iota(jnp.int32, Lbig.shape, 1) // Ns
    Lm = jnp.where(rb == cb, jnp.tanh(Lbig), 0.0)

    XBWlT = jax.lax.dot_general(XB, Wl_ref[...], cn,
                                preferred_element_type=jnp.float32)
    LbigT = jax.lax.dot_general(XBWlT, XD, cn,
                                preferred_element_type=jnp.float32)
    rbT = jax.lax.broadcasted_iota(jnp.int32, LbigT.shape, 0) // Ns
    cbT = jax.lax.broadcasted_iota(jnp.int32, LbigT.shape, 1) // Nc
    LmT = jnp.where(rbT == cbT, jnp.tanh(LbigT), 0.0)

    S_b = jax.lax.dot_general(XB, Ws_ref[...], cn,
                              preferred_element_type=jnp.float32)
    C_b = jax.lax.dot_general(XD, Wc_ref[...], cn,
                              preferred_element_type=jnp.float32)
    HsT = jnp.tanh(S_b + jnp.dot(LmT, C_b, preferred_element_type=jnp.float32))
    HcT = jnp.tanh(C_b + jnp.dot(Lm, S_b, preferred_element_type=jnp.float32))

    vs = jnp.sum(HsT * whs_ref[...], axis=1, keepdims=True).reshape(TB, Ns, 1)
    As = _softmax_ax1(vs)
    As_ref[...] = As
    co_s = jnp.sum(XB3 * As, axis=1)

    vc = jnp.sum(HcT * whc_ref[...], axis=1, keepdims=True).reshape(TB, Nc, 1)
    Ac = _softmax_ax1(vc)
    Ac_ref[...] = Ac
    co_c = jnp.sum(XD3 * Ac, axis=1)

    cat = jnp.concatenate([xd_a, co_s, co_c], axis=1)
    logits = (jnp.dot(cat, Wfc_ref[...], preferred_element_type=jnp.float32)
              + bfc_ref[...])
    m = jnp.max(logits, axis=1, keepdims=True)
    e = jnp.exp(logits - m)
    preds_ref[...] = e / jnp.sum(e, axis=1, keepdims=True)


def _head_pass(xb, xd, W_att, b_att, u_att, Wl, Wc, Ws, whs, whc, Wfc, bfc):
    B, Ns, _ = xb.shape
    Nc = xd.shape[1]
    TB = 16
    while B % TB:
        TB //= 2

    W_att_p = jnp.pad(W_att, ((0, 0), (0, ATT_PAD - ATT_DIM)))
    b_att_p = jnp.pad(b_att, ((0, 0), (0, ATT_PAD - ATT_DIM)))
    u_row = jnp.pad(u_att.T, ((0, 0), (0, ATT_PAD - ATT_DIM)))
    Wc_p = jnp.pad(Wc, ((0, K_PAD - K_CO), (0, 0)))
    Ws_p = jnp.pad(Ws, ((0, K_PAD - K_CO), (0, 0)))
    whs_r = jnp.pad(whs, ((0, 0), (0, K_PAD - K_CO)))
    whc_r = jnp.pad(whc, ((0, 0), (0, K_PAD - K_CO)))

    def full(shape):
        return pl.BlockSpec(shape, lambda b, _n=len(shape): (0,) * _n)

    out_shape = (
        jax.ShapeDtypeStruct((B, NUM_CLASSES), jnp.float32),
        jax.ShapeDtypeStruct((B, Ns, 1), jnp.float32),
        jax.ShapeDtypeStruct((B, Nc, 1), jnp.float32),
        jax.ShapeDtypeStruct((B, Nc, 1), jnp.float32),
    )
    in_specs = [
        pl.BlockSpec((TB, Ns, LATENT), lambda b: (b, 0, 0)),
        pl.BlockSpec((TB, Nc, LATENT), lambda b: (b, 0, 0)),
        full((LATENT, ATT_PAD)),
        full((1, ATT_PAD)),
        full((1, ATT_PAD)),
        full((LATENT, LATENT)),
        full((K_PAD, LATENT)),
        full((K_PAD, LATENT)),
        full((1, K_PAD)),
        full((1, K_PAD)),
        full((3 * LATENT, NUM_CLASSES)),
        full((1, NUM_CLASSES)),
    ]
    out_specs = (
        pl.BlockSpec((TB, NUM_CLASSES), lambda b: (b, 0)),
        pl.BlockSpec((TB, Ns, 1), lambda b: (b, 0, 0)),
        pl.BlockSpec((TB, Nc, 1), lambda b: (b, 0, 0)),
        pl.BlockSpec((TB, Nc, 1), lambda b: (b, 0, 0)),
    )
    return pl.pallas_call(
        _head_kernel,
        out_shape=out_shape,
        grid=(B // TB,),
        in_specs=in_specs,
        out_specs=out_specs,
        compiler_params=pltpu.CompilerParams(dimension_semantics=("parallel",)),
    )(xb, xd, W_att_p, b_att_p, u_row, Wl, Wc_p, Ws_p, whs_r, whc_r, Wfc, bfc)


def kernel(content, comments, emo_emb, w1, b1, w2, b2, W_att, b_att, u_att,
           Wl, Wc, Ws, whs, whc, Wfc, bfc):
    B, NS, S = content.shape
    NC = comments.shape[1]

    wg = _build_group_weights(w1, w2)
    b_fused = jnp.concatenate([b1, b2], axis=1)

    xb = _conv_features(content.reshape(B * NS, S), emo_emb, wg,
                        b_fused).reshape(B, NS, LATENT)
    xd = _conv_features(comments.reshape(B * NC, S), emo_emb, wg,
                        b_fused).reshape(B, NC, LATENT)
    return _head_pass(xb, xd, W_att, b_att, u_att, Wl, Wc, Ws, whs, whc, Wfc, bfc)


# speedup vs baseline: 2.8774x; 1.0552x over previous
import jax
import jax.numpy as jnp
from jax.experimental import pallas as pl
from jax.experimental.pallas import tpu as pltpu

EMB = 32
SEQ = 16
CONV_OUT = 64
LATENT = 2 * CONV_OUT
ATT_DIM = 100
ATT_PAD = 128
K_CO = 80
K_PAD = 128
NUM_CLASSES = 2
EPS = 1e-7

TM = 1024
_CHUNK_ROWS = TM * SEQ // 128

_GROUPS = ((0, (0, 1, 2, 3, 4, 5)),
           (4, (6, 7, 8, 9)),
           (8, (10, 11, 12, 13, 14)))


def _round_up(a, b):
    return (a + b - 1) // b * b


def _conv_kernel(idx_ref, t0_ref, t1_ref, t2_ref, t3_ref,
                 wg0_ref, wg1_ref, wg2_ref, b_ref, out_ref,
                 x_scratch, idx_smem, sem):
    tbls = (t0_ref, t1_ref, t2_ref, t3_ref)

    QR = _CHUNK_ROWS // 4
    cps = []
    for qtr in range(4):
        cp = pltpu.make_async_copy(idx_ref.at[pl.ds(qtr * QR, QR), :],
                                   idx_smem.at[pl.ds(qtr * QR, QR), :],
                                   sem.at[qtr])
        cp.start()
        cps.append(cp)

    for qtr in range(4):
        cps[qtr].wait()
        for c in range(qtr * QR, (qtr + 1) * QR):
            for m in range(8):
                for g in range(4):
                    k0 = m * SEQ + g * 4
                    p = (tbls[0][idx_smem[c, k0 + 0]]
                         + tbls[1][idx_smem[c, k0 + 1]])
                    q = (tbls[2][idx_smem[c, k0 + 2]]
                         + tbls[3][idx_smem[c, k0 + 3]])
                    r = c * 8 + m
                    x_scratch[r:r + 1, g * 128:(g + 1) * 128] = p + q

    x = x_scratch[...]
    ys = (
        jnp.dot(x[:, 0:256], wg0_ref[...], preferred_element_type=jnp.float32),
        jnp.dot(x[:, 128:384], wg1_ref[...], preferred_element_type=jnp.float32),
        jnp.dot(x[:, 256:512], wg2_ref[...], preferred_element_type=jnp.float32),
    )
    chunks = []
    for y, (_, ps) in zip(ys, _GROUPS):
        for i in range(len(ps)):
            chunks.append(y[:, i * LATENT:(i + 1) * LATENT])
    lane = jax.lax.broadcasted_iota(jnp.int32, chunks[-1].shape, 1)
    chunks[-1] = jnp.where(lane < CONV_OUT, chunks[-1], -jnp.inf)
    while len(chunks) > 1:
        nxt = [jnp.maximum(chunks[i], chunks[i + 1])
               for i in range(0, len(chunks) - 1, 2)]
        if len(chunks) % 2:
            nxt.append(chunks[-1])
        chunks = nxt
    out_ref[...] = jnp.maximum(chunks[0] + b_ref[...], 0.0)


def _build_group_weights(w1, w2):
    taps = [jnp.concatenate(
        [w1[t] if t < 2 else jnp.zeros((EMB, CONV_OUT), jnp.float32), w2[t]],
        axis=1) for t in range(3)]
    outs = []
    for base, ps in _GROUPS:
        W = jnp.zeros((8 * EMB, len(ps) * LATENT), jnp.float32)
        for i, p in enumerate(ps):
            for t in range(3):
                if p + t >= SEQ:
                    continue
                o = p - base + t
                W = W.at[o * EMB:(o + 1) * EMB, i * LATENT:(i + 1) * LATENT].set(taps[t])
        outs.append(W)
    return outs


def _conv_features(idx, emo_emb, wg, b_fused):
    N = idx.shape[0]
    N_pad = _round_up(N, TM)
    if N_pad != N:
        idx = jnp.pad(idx, ((0, N_pad - N), (0, 0)))
    idx2d = idx.reshape(N_pad * SEQ // 128, 128)

    V = emo_emb.shape[0]
    tbls = []
    for r in range(4):
        t = jnp.zeros((V, 128), jnp.float32).at[:, r * EMB:(r + 1) * EMB].set(emo_emb)
        tbls.append(t.reshape(V, 1, 128))

    out = pl.pallas_call(
        _conv_kernel,
        out_shape=jax.ShapeDtypeStruct((N_pad, LATENT), jnp.float32),
        grid=(N_pad // TM,),
        in_specs=[
            pl.BlockSpec((_CHUNK_ROWS, 128), lambda n: (n, 0)),
            pl.BlockSpec((V, 1, 128), lambda n: (0, 0, 0)),
            pl.BlockSpec((V, 1, 128), lambda n: (0, 0, 0)),
            pl.BlockSpec((V, 1, 128), lambda n: (0, 0, 0)),
            pl.BlockSpec((V, 1, 128), lambda n: (0, 0, 0)),
            pl.BlockSpec(wg[0].shape, lambda n: (0, 0)),
            pl.BlockSpec(wg[1].shape, lambda n: (0, 0)),
            pl.BlockSpec(wg[2].shape, lambda n: (0, 0)),
            pl.BlockSpec((1, LATENT), lambda n: (0, 0)),
        ],
        out_specs=pl.BlockSpec((TM, LATENT), lambda n: (n, 0)),
        scratch_shapes=[
            pltpu.VMEM((TM, SEQ * EMB), jnp.float32),
            pltpu.SMEM((_CHUNK_ROWS, 128), jnp.int32),
            pltpu.SemaphoreType.DMA((4,)),
        ],
        compiler_params=pltpu.CompilerParams(
            dimension_semantics=("parallel",),
            vmem_limit_bytes=56 * 1024 * 1024,
        ),
    )(idx2d, tbls[0], tbls[1], tbls[2], tbls[3], wg[0], wg[1], wg[2], b_fused)
    return out[:N]


def _softmax_ax1(x):
    m = jnp.max(x, axis=1, keepdims=True)
    e = jnp.exp(x - m)
    return e / jnp.sum(e, axis=1, keepdims=True)


def _head_kernel(xb_ref, xd_ref, Watt_ref, batt_ref, u_ref, Wl_ref, Wc_ref,
                 Ws_ref, whs_ref, whc_ref, Wfc_ref, bfc_ref,
                 preds_ref, As_ref, Ac_ref, ait_ref):
    TB, Ns, _ = xb_ref.shape
    Nc = xd_ref.shape[1]
    XB3 = xb_ref[...]
    XD3 = xd_ref[...]
    XB = XB3.reshape(TB * Ns, LATENT)
    XD = XD3.reshape(TB * Nc, LATENT)
    cn = (((1,), (1,)), ((), ()))

    uit = jnp.tanh(jnp.dot(XD, Watt_ref[...], preferred_element_type=jnp.float32)
                   + batt_ref[...])
    s = jnp.sum(uit.reshape(TB, Nc, LATENT) * u_ref[...].reshape(1, 1, LATENT),
                axis=2, keepdims=True)
    a = jnp.exp(s)
    an = a / (jnp.sum(a, axis=1, keepdims=True) + EPS)
    ait_ref[...] = an
    xd_a = jnp.sum(XD3 * an, axis=1)

    XDWl = jnp.dot(XD, Wl_ref[...], preferred_element_type=jnp.float32)
    Lbig = jax.lax.dot_general(XDWl, XB, cn,
                               preferred_element_type=jnp.float32)
    rb = jax.lax.broadcasted_iota(jnp.int32, Lbig.shape, 0) // Nc
    cb = jax.lax.broadcasted_iota(jnp.int32, Lbig.shape, 1) // Ns
    Lm = jnp.where(rb == cb, jnp.tanh(Lbig), 0.0)

    XBWlT = jax.lax.dot_general(XB, Wl_ref[...], cn,
                                preferred_element_type=jnp.float32)
    LbigT = jax.lax.dot_general(XBWlT, XD, cn,
                                preferred_element_type=jnp.float32)
    rbT = jax.lax.broadcasted_iota(jnp.int32, LbigT.shape, 0) // Ns
    cbT = jax.lax.broadcasted_iota(jnp.int32, LbigT.shape, 1) // Nc
    LmT = jnp.where(rbT == cbT, jnp.tanh(LbigT), 0.0)

    S_b = jax.lax.dot_general(XB, Ws_ref[...], cn,
                              preferred_element_type=jnp.float32)
    C_b = jax.lax.dot_general(XD, Wc_ref[...], cn,
                              preferred_element_type=jnp.float32)
    HsT = jnp.tanh(S_b + jnp.dot(LmT, C_b, preferred_element_type=jnp.float32))
    HcT = jnp.tanh(C_b + jnp.dot(Lm, S_b, preferred_element_type=jnp.float32))

    vs = jnp.sum(HsT * whs_ref[...], axis=1, keepdims=True).reshape(TB, Ns, 1)
    As = _softmax_ax1(vs)
    As_ref[...] = As
    co_s = jnp.sum(XB3 * As, axis=1)

    vc = jnp.sum(HcT * whc_ref[...], axis=1, keepdims=True).reshape(TB, Nc, 1)
    Ac = _softmax_ax1(vc)
    Ac_ref[...] = Ac
    co_c = jnp.sum(XD3 * Ac, axis=1)

    cat = jnp.concatenate([xd_a, co_s, co_c], axis=1)
    logits = (jnp.dot(cat, Wfc_ref[...], preferred_element_type=jnp.float32)
              + bfc_ref[...])
    m = jnp.max(logits, axis=1, keepdims=True)
    e = jnp.exp(logits - m)
    preds_ref[...] = e / jnp.sum(e, axis=1, keepdims=True)


def _head_pass(xb, xd, W_att, b_att, u_att, Wl, Wc, Ws, whs, whc, Wfc, bfc):
    B, Ns, _ = xb.shape
    Nc = xd.shape[1]
    TB = 16
    while B % TB:
        TB //= 2

    W_att_p = jnp.pad(W_att, ((0, 0), (0, ATT_PAD - ATT_DIM)))
    b_att_p = jnp.pad(b_att, ((0, 0), (0, ATT_PAD - ATT_DIM)))
    u_row = jnp.pad(u_att.T, ((0, 0), (0, ATT_PAD - ATT_DIM)))
    Wc_p = jnp.pad(Wc, ((0, K_PAD - K_CO), (0, 0)))
    Ws_p = jnp.pad(Ws, ((0, K_PAD - K_CO), (0, 0)))
    whs_r = jnp.pad(whs, ((0, 0), (0, K_PAD - K_CO)))
    whc_r = jnp.pad(whc, ((0, 0), (0, K_PAD - K_CO)))

    def full(shape):
        return pl.BlockSpec(shape, lambda b, _n=len(shape): (0,) * _n)

    out_shape = (
        jax.ShapeDtypeStruct((B, NUM_CLASSES), jnp.float32),
        jax.ShapeDtypeStruct((B, Ns, 1), jnp.float32),
        jax.ShapeDtypeStruct((B, Nc, 1), jnp.float32),
        jax.ShapeDtypeStruct((B, Nc, 1), jnp.float32),
    )
    in_specs = [
        pl.BlockSpec((TB, Ns, LATENT), lambda b: (b, 0, 0)),
        pl.BlockSpec((TB, Nc, LATENT), lambda b: (b, 0, 0)),
        full((LATENT, ATT_PAD)),
        full((1, ATT_PAD)),
        full((1, ATT_PAD)),
        full((LATENT, LATENT)),
        full((K_PAD, LATENT)),
        full((K_PAD, LATENT)),
        full((1, K_PAD)),
        full((1, K_PAD)),
        full((3 * LATENT, NUM_CLASSES)),
        full((1, NUM_CLASSES)),
    ]
    out_specs = (
        pl.BlockSpec((TB, NUM_CLASSES), lambda b: (b, 0)),
        pl.BlockSpec((TB, Ns, 1), lambda b: (b, 0, 0)),
        pl.BlockSpec((TB, Nc, 1), lambda b: (b, 0, 0)),
        pl.BlockSpec((TB, Nc, 1), lambda b: (b, 0, 0)),
    )
    return pl.pallas_call(
        _head_kernel,
        out_shape=out_shape,
        grid=(B // TB,),
        in_specs=in_specs,
        out_specs=out_specs,
        compiler_params=pltpu.CompilerParams(dimension_semantics=("parallel",)),
    )(xb, xd, W_att_p, b_att_p, u_row, Wl, Wc_p, Ws_p, whs_r, whc_r, Wfc, bfc)


def kernel(content, comments, emo_emb, w1, b1, w2, b2, W_att, b_att, u_att,
           Wl, Wc, Ws, whs, whc, Wfc, bfc):
    B, NS, S = content.shape
    NC = comments.shape[1]

    wg = _build_group_weights(w1, w2)
    b_fused = jnp.concatenate([b1, b2], axis=1)

    xb = _conv_features(content.reshape(B * NS, S), emo_emb, wg,
                        b_fused).reshape(B, NS, LATENT)
    xd = _conv_features(comments.reshape(B * NC, S), emo_emb, wg,
                        b_fused).reshape(B, NC, LATENT)
    return _head_pass(xb, xd, W_att, b_att, u_att, Wl, Wc, Ws, whs, whc, Wfc, bfc)
